# Initial kernel scaffold; baseline (speedup 1.0000x reference)
import jax
import jax.numpy as jnp
from jax.experimental import pallas as pl


def kernel(x, edge_index_0, edge_index_1, index_0, index_1, batch, W1_0, b1_0, W1_1, b1_1, W2_0, b2_0, W2_1, b2_1, mlp1_W1, mlp1_b1, mlp1_W2, mlp1_b2, mlp2_W1, mlp2_b1, mlp2_W2, mlp2_b2, lin_W, lin_b):
    raise NotImplementedError("write your pallas kernel here")


# speedup vs baseline: 36.4631x; 36.4631x over previous
import functools

import jax
import jax.numpy as jnp
from jax import lax
from jax.experimental import pallas as pl
from jax.experimental.pallas import tpu as pltpu
from jax.experimental.pallas import tpu_sc as plsc

N = 50000
E = 1600000
D_IN = 128
H = 32
G = 128

NC, NS = 2, 16
N_PAD = 53248
E_PAD = 1605632
ROWS_E = E_PAD // 128
ROWS_PT = ROWS_E // NS
BLK = 4
NBLK = ROWS_PT // BLK
GP = 136
ROWS_N = N_PAD // 128
ROWS_N_PT = ROWS_N // (NC * NS)
RPT = N_PAD // NS

_MESH = plsc.VectorSubcoreMesh(core_axis_name="c", subcore_axis_name="s")
_SC_PARAMS = pltpu.CompilerParams(use_tc_tiling_on_sc=False)


def _hist_body(dst_hbm, zeros_hbm, out_hbm, idx_v, ones_v, acc_sh):
    c = lax.axis_index("c")
    s = lax.axis_index("s")
    for i in range(8):
        ones_v[pl.ds(16 * i, 16)] = jnp.full((16,), 1.0, jnp.float32)
    pltpu.sync_copy(zeros_hbm.at[pl.ds(s * RPT, RPT)],
                    acc_sh.at[pl.ds(s * RPT, RPT)])
    plsc.subcore_barrier()
    base = s * ROWS_PT

    def blk(i, carry):
        pltpu.sync_copy(dst_hbm.at[c, pl.ds(base + i * BLK, BLK)], idx_v)
        for j in range(BLK):
            pltpu.sync_copy(ones_v, acc_sh.at[idx_v.at[j]], add=True)
        return carry

    lax.fori_loop(0, NBLK, blk, 0)
    plsc.subcore_barrier()
    pltpu.sync_copy(acc_sh.at[pl.ds(s * RPT, RPT)],
                    out_hbm.at[c, pl.ds(s * RPT, RPT)])


_hist = pl.kernel(
    _hist_body,
    out_type=jax.ShapeDtypeStruct((NC, N_PAD), jnp.float32),
    mesh=_MESH,
    compiler_params=_SC_PARAMS,
    scratch_types=[
        pltpu.VMEM((BLK, 128), jnp.int32),
        pltpu.VMEM((128,), jnp.float32),
        pltpu.VMEM_SHARED((N_PAD,), jnp.float32),
    ],
)


def _conv_body(src_hbm, dst_hbm, h_hbm, zeros_hbm, out_hbm,
               sidx_v, didx_v, rows_v, sem, acc_sh):
    c = lax.axis_index("c")
    s = lax.axis_index("s")
    pltpu.sync_copy(zeros_hbm.at[pl.ds(s * RPT, RPT)],
                    acc_sh.at[pl.ds(s * RPT, RPT)])
    plsc.subcore_barrier()
    ebase = s * (ROWS_PT * 128)
    rbase = s * ROWS_PT

    def blk(i, carry):
        pltpu.sync_copy(src_hbm.at[c, pl.ds(ebase + i * (BLK * 128), BLK * 128)],
                        sidx_v)
        pltpu.sync_copy(dst_hbm.at[c, pl.ds(rbase + i * BLK, BLK)], didx_v)
        pltpu.async_copy(h_hbm.at[sidx_v], rows_v, sem).wait()
        for j in range(BLK):
            pltpu.sync_copy(rows_v.at[pl.ds(j * 128, 128)],
                            acc_sh.at[didx_v.at[j]], add=True)
        return carry

    lax.fori_loop(0, NBLK, blk, 0)
    plsc.subcore_barrier()
    pltpu.sync_copy(acc_sh.at[pl.ds(s * RPT, RPT)],
                    out_hbm.at[c, pl.ds(s * RPT, RPT)])


_conv = pl.kernel(
    _conv_body,
    out_type=jax.ShapeDtypeStruct((NC, N_PAD, H), jnp.float32),
    mesh=_MESH,
    compiler_params=_SC_PARAMS,
    scratch_types=[
        pltpu.VMEM((BLK * 128,), jnp.int32),
        pltpu.VMEM((BLK, 128), jnp.int32),
        pltpu.VMEM((BLK * 128, H), jnp.float32),
        pltpu.SemaphoreType.DMA,
        pltpu.VMEM_SHARED((N_PAD, H), jnp.float32),
    ],
)


def _pool_body(xm2_hbm, batch_hbm, zeros_hbm, out_hbm, bidx_v, rows_v, acc_sh):
    c = lax.axis_index("c")
    s = lax.axis_index("s")
    w = s * NC + c

    @pl.when(s == 0)
    def _():
        pltpu.sync_copy(zeros_hbm, acc_sh)

    plsc.subcore_barrier()
    pltpu.sync_copy(batch_hbm.at[pl.ds(w * ROWS_N_PT, ROWS_N_PT)], bidx_v)

    def blk(i, carry):
        pltpu.sync_copy(xm2_hbm.at[pl.ds((w * ROWS_N_PT + i) * 128, 128)],
                        rows_v)
        pltpu.sync_copy(rows_v, acc_sh.at[bidx_v.at[i]], add=True)
        return carry

    lax.fori_loop(0, ROWS_N_PT, blk, 0)
    plsc.subcore_barrier()

    @pl.when(s == 0)
    def _():
        pltpu.sync_copy(acc_sh, out_hbm.at[c])


_pool = pl.kernel(
    _pool_body,
    out_type=jax.ShapeDtypeStruct((NC, GP, H), jnp.float32),
    mesh=_MESH,
    compiler_params=_SC_PARAMS,
    scratch_types=[
        pltpu.VMEM((ROWS_N_PT, 128), jnp.int32),
        pltpu.VMEM((128, H), jnp.float32),
        pltpu.VMEM_SHARED((GP, H), jnp.float32),
    ],
)


R = 2048
GRID = N_PAD // R
_f32 = jnp.float32


def _dense_a_body(x_ref, w_ref, dg0_ref, dg1_ref,
                  h0_ref, h1_ref, v0_ref, v1_ref):
    m = jnp.dot(x_ref[...], w_ref[...], preferred_element_type=_f32)
    v0 = lax.rsqrt(dg0_ref[...] + 1.0)
    v1 = lax.rsqrt(dg1_ref[...] + 1.0)
    h0_ref[...] = m[:, :H] * v0
    h1_ref[...] = m[:, H:] * v1
    v0_ref[...] = v0
    v1_ref[...] = v1


_dense_a = pl.pallas_call(
    _dense_a_body,
    grid=(GRID,),
    in_specs=[
        pl.BlockSpec((R, D_IN), lambda i: (i, 0)),
        pl.BlockSpec((D_IN, 2 * H), lambda i: (0, 0)),
        pl.BlockSpec((R, 1), lambda i: (i, 0)),
        pl.BlockSpec((R, 1), lambda i: (i, 0)),
    ],
    out_specs=[
        pl.BlockSpec((R, H), lambda i: (i, 0)),
        pl.BlockSpec((R, H), lambda i: (i, 0)),
        pl.BlockSpec((R, 1), lambda i: (i, 0)),
        pl.BlockSpec((R, 1), lambda i: (i, 0)),
    ],
    out_shape=[
        jax.ShapeDtypeStruct((N_PAD, H), _f32),
        jax.ShapeDtypeStruct((N_PAD, H), _f32),
        jax.ShapeDtypeStruct((N_PAD, 1), _f32),
        jax.ShapeDtypeStruct((N_PAD, 1), _f32),
    ],
)


def _dense_b_body(s0_ref, s1_ref, h0p_ref, h1p_ref, v0_ref, v1_ref,
                  b10_ref, b11_ref, m1a_ref, m1b_ref, mb1_ref,
                  m1w2_ref, mb2_ref, w20_ref, w21_ref,
                  g0_ref, g1_ref):
    v0 = v0_ref[...]
    v1 = v1_ref[...]
    h0 = jnp.maximum(v0 * (s0_ref[...] + h0p_ref[...]) + b10_ref[...], 0.0)
    h1 = jnp.maximum(v1 * (s1_ref[...] + h1p_ref[...]) + b11_ref[...], 0.0)
    t = jnp.maximum(
        jnp.dot(h0, m1a_ref[...], preferred_element_type=_f32)
        + jnp.dot(h1, m1b_ref[...], preferred_element_type=_f32)
        + mb1_ref[...], 0.0)
    xm = jnp.dot(t, m1w2_ref[...], preferred_element_type=_f32) + mb2_ref[...]
    g0_ref[...] = jnp.dot(xm, w20_ref[...], preferred_element_type=_f32) * v0
    g1_ref[...] = jnp.dot(xm, w21_ref[...], preferred_element_type=_f32) * v1


def _row_spec():
    return pl.BlockSpec((R, H), lambda i: (i, 0))


def _one_spec():
    return pl.BlockSpec((R, 1), lambda i: (i, 0))


def _mat_spec():
    return pl.BlockSpec((H, H), lambda i: (0, 0))


def _bias_spec():
    return pl.BlockSpec((1, H), lambda i: (0, 0))


_dense_b = pl.pallas_call(
    _dense_b_body,
    grid=(GRID,),
    in_specs=[
        _row_spec(), _row_spec(), _row_spec(), _row_spec(),
        _one_spec(), _one_spec(),
        _bias_spec(), _bias_spec(),
        _mat_spec(), _mat_spec(), _bias_spec(),
        _mat_spec(), _bias_spec(),
        _mat_spec(), _mat_spec(),
    ],
    out_specs=[_row_spec(), _row_spec()],
    out_shape=[
        jax.ShapeDtypeStruct((N_PAD, H), _f32),
        jax.ShapeDtypeStruct((N_PAD, H), _f32),
    ],
)


def _dense_c_body(s0_ref, s1_ref, g0p_ref, g1p_ref, v0_ref, v1_ref,
                  b20_ref, b21_ref, m2a_ref, m2b_ref, mb1_ref,
                  m2w2_ref, mb2_ref, xm2_ref):
    g0 = jnp.maximum(v0_ref[...] * (s0_ref[...] + g0p_ref[...]) + b20_ref[...],
                     0.0)
    g1 = jnp.maximum(v1_ref[...] * (s1_ref[...] + g1p_ref[...]) + b21_ref[...],
                     0.0)
    t = jnp.maximum(
        jnp.dot(g0, m2a_ref[...], preferred_element_type=_f32)
        + jnp.dot(g1, m2b_ref[...], preferred_element_type=_f32)
        + mb1_ref[...], 0.0)
    xm2_ref[...] = (jnp.dot(t, m2w2_ref[...], preferred_element_type=_f32)
                    + mb2_ref[...])


_dense_c = pl.pallas_call(
    _dense_c_body,
    grid=(GRID,),
    in_specs=[
        _row_spec(), _row_spec(), _row_spec(), _row_spec(),
        _one_spec(), _one_spec(),
        _bias_spec(), _bias_spec(),
        _mat_spec(), _mat_spec(), _bias_spec(),
        _mat_spec(), _bias_spec(),
    ],
    out_specs=[_row_spec()],
    out_shape=[jax.ShapeDtypeStruct((N_PAD, H), _f32)],
)


def _dense_d_body(p_ref, w_ref, b_ref, o_ref):
    p = p_ref[0, :G, :] + p_ref[1, :G, :]
    o_ref[...] = jnp.dot(p, w_ref[...], preferred_element_type=_f32) + b_ref[...]


_dense_d = pl.pallas_call(
    _dense_d_body,
    grid=(1,),
    in_specs=[
        pl.BlockSpec((NC, GP, H), lambda i: (0, 0, 0)),
        pl.BlockSpec((H, 1), lambda i: (0, 0)),
        pl.BlockSpec((1, 1), lambda i: (0, 0)),
    ],
    out_specs=pl.BlockSpec((G, 1), lambda i: (0, 0)),
    out_shape=jax.ShapeDtypeStruct((G, 1), _f32),
)


def kernel(x, edge_index_0, edge_index_1, index_0, index_1, batch,
           W1_0, b1_0, W1_1, b1_1, W2_0, b2_0, W2_1, b2_1,
           mlp1_W1, mlp1_b1, mlp1_W2, mlp1_b2,
           mlp2_W1, mlp2_b1, mlp2_W2, mlp2_b2,
           lin_W, lin_b):
    i32 = jnp.int32
    x_p = jnp.zeros((N_PAD, D_IN), _f32).at[:N].set(x)
    pad_idx = jnp.full((E_PAD - E,), N_PAD - 1, i32)
    s0 = jnp.concatenate([edge_index_0[0].astype(i32), pad_idx])
    d0 = jnp.concatenate([edge_index_0[1].astype(i32), pad_idx])
    s1 = jnp.concatenate([edge_index_1[0].astype(i32), pad_idx])
    d1 = jnp.concatenate([edge_index_1[1].astype(i32), pad_idx])
    src_all = jnp.stack([s0, s1 + N_PAD])
    dst_all = jnp.stack([d0, d1]).reshape(NC, ROWS_E, 128)
    batch_p = jnp.concatenate(
        [batch.astype(i32), jnp.full((N_PAD - N,), G, i32)]).reshape(ROWS_N, 128)
    zeros_nh = jnp.zeros((N_PAD, H), _f32)
    zeros_n = jnp.zeros((N_PAD,), _f32)
    zeros_g = jnp.zeros((GP, H), _f32)

    deg = _hist(dst_all, zeros_n)
    h0p, h1p, dv0, dv1 = _dense_a(
        x_p, jnp.concatenate([W1_0, W1_1], axis=1),
        deg[0].reshape(N_PAD, 1), deg[1].reshape(N_PAD, 1))

    hs = jnp.concatenate([h0p, h1p], axis=0)
    sc1 = _conv(src_all, dst_all, hs, zeros_nh)

    g0p, g1p = _dense_b(
        sc1[0], sc1[1], h0p, h1p, dv0, dv1,
        b1_0.reshape(1, H), b1_1.reshape(1, H),
        mlp1_W1[:H], mlp1_W1[H:], mlp1_b1.reshape(1, H),
        mlp1_W2, mlp1_b2.reshape(1, H),
        W2_0, W2_1)

    gs = jnp.concatenate([g0p, g1p], axis=0)
    sc2 = _conv(src_all, dst_all, gs, zeros_nh)

    (xm2,) = _dense_c(
        sc2[0], sc2[1], g0p, g1p, dv0, dv1,
        b2_0.reshape(1, H), b2_1.reshape(1, H),
        mlp2_W1[:H], mlp2_W1[H:], mlp2_b1.reshape(1, H),
        mlp2_W2, mlp2_b2.reshape(1, H))

    pooled = _pool(xm2, batch_p, zeros_g)
    out = _dense_d(pooled, lin_W, lin_b.reshape(1, 1))
    return out.reshape(G)


# speedup vs baseline: 41.5196x; 1.1387x over previous
import functools

import jax
import jax.numpy as jnp
from jax import lax
from jax.experimental import pallas as pl
from jax.experimental.pallas import tpu as pltpu
from jax.experimental.pallas import tpu_sc as plsc

N = 50000
E = 1600000
D_IN = 128
H = 32
G = 128

NC, NS = 2, 16
N_PAD = 53248
E_PAD = 1605632
ROWS_E = E_PAD // 128
ROWS_PT = ROWS_E // NS
BLK = 4
NBLK = ROWS_PT // BLK
GP = 136
ROWS_N = N_PAD // 128
ROWS_N_PT = ROWS_N // (NC * NS)
RPT = N_PAD // NS

_MESH = plsc.VectorSubcoreMesh(core_axis_name="c", subcore_axis_name="s")
_SC_PARAMS = pltpu.CompilerParams(use_tc_tiling_on_sc=False)


HB = 2048
NHB = (E_PAD // NS) // HB


def _hist_body(dst_hbm, zeros_hbm, out_hbm, idx_v, ones_v, acc_sh):
    c = lax.axis_index("c")
    s = lax.axis_index("s")

    def fill(i, carry):
        ones_v[pl.ds(i * 16, 16)] = jnp.full((16,), 1.0, jnp.float32)
        return carry

    lax.fori_loop(0, HB // 16, fill, 0)
    pltpu.sync_copy(zeros_hbm.at[pl.ds(s * RPT, RPT)],
                    acc_sh.at[pl.ds(s * RPT, RPT)])
    plsc.subcore_barrier()
    ebase = s * (E_PAD // NS)

    def blk(i, carry):
        pltpu.sync_copy(dst_hbm.at[c, pl.ds(ebase + i * HB, HB)], idx_v)
        pltpu.sync_copy(ones_v, acc_sh.at[idx_v], add=True)
        return carry

    lax.fori_loop(0, NHB, blk, 0)
    plsc.subcore_barrier()
    pltpu.sync_copy(acc_sh.at[pl.ds(s * RPT, RPT)],
                    out_hbm.at[c, pl.ds(s * RPT, RPT)])


_hist = pl.kernel(
    _hist_body,
    out_type=jax.ShapeDtypeStruct((NC, N_PAD), jnp.float32),
    mesh=_MESH,
    compiler_params=_SC_PARAMS,
    scratch_types=[
        pltpu.VMEM((HB,), jnp.int32),
        pltpu.VMEM((HB,), jnp.float32),
        pltpu.VMEM_SHARED((N_PAD,), jnp.float32),
    ],
)


CB = 256
NCB = (E_PAD // NS) // CB


def _conv_body(src_hbm, dst_hbm, h_hbm, zeros_hbm, out_hbm,
               sidx0, sidx1, didx0, didx1, rows0, rows1, gsem0, gsem1,
               acc_sh):
    c = lax.axis_index("c")
    s = lax.axis_index("s")
    pltpu.sync_copy(zeros_hbm.at[pl.ds(s * RPT, RPT)],
                    acc_sh.at[pl.ds(s * RPT, RPT)])
    plsc.subcore_barrier()
    ebase = s * (E_PAD // NS)

    sidx = (sidx0, sidx1)
    didx = (didx0, didx1)
    rows = (rows0, rows1)
    gsem = (gsem0, gsem1)

    def stage_and_gather(b, buf):
        off = ebase + b * CB
        pltpu.sync_copy(src_hbm.at[c, pl.ds(off, CB)], sidx[buf])
        pltpu.sync_copy(dst_hbm.at[c, pl.ds(off, CB)], didx[buf])
        pltpu.make_async_copy(h_hbm.at[sidx[buf]], rows[buf],
                              gsem[buf]).start()

    stage_and_gather(0, 0)

    def blk(i, carry):
        for half in range(2):
            b = 2 * i + half
            buf = half
            nbuf = 1 - half
            nb = b + 1

            @pl.when(nb < NCB)
            def _():
                stage_and_gather(nb, nbuf)

            pltpu.make_async_copy(h_hbm.at[sidx[buf]], rows[buf],
                                  gsem[buf]).wait()
            pltpu.sync_copy(rows[buf], acc_sh.at[didx[buf]], add=True)
        return carry

    lax.fori_loop(0, NCB // 2, blk, 0)
    plsc.subcore_barrier()
    pltpu.sync_copy(acc_sh.at[pl.ds(s * RPT, RPT)],
                    out_hbm.at[c, pl.ds(s * RPT, RPT)])


_conv = pl.kernel(
    _conv_body,
    out_type=jax.ShapeDtypeStruct((NC, N_PAD, H), jnp.float32),
    mesh=_MESH,
    compiler_params=_SC_PARAMS,
    scratch_types=[
        pltpu.VMEM((CB,), jnp.int32),
        pltpu.VMEM((CB,), jnp.int32),
        pltpu.VMEM((CB,), jnp.int32),
        pltpu.VMEM((CB,), jnp.int32),
        pltpu.VMEM((CB, H), jnp.float32),
        pltpu.VMEM((CB, H), jnp.float32),
        pltpu.SemaphoreType.DMA,
        pltpu.SemaphoreType.DMA,
        pltpu.VMEM_SHARED((N_PAD, H), jnp.float32),
    ],
)


def _pool_body(xm2_hbm, batch_hbm, zeros_hbm, out_hbm, bidx_v, rows_v, acc_sh):
    c = lax.axis_index("c")
    s = lax.axis_index("s")
    w = s * NC + c

    @pl.when(s == 0)
    def _():
        pltpu.sync_copy(zeros_hbm, acc_sh)

    plsc.subcore_barrier()
    pltpu.sync_copy(batch_hbm.at[pl.ds(w * ROWS_N_PT, ROWS_N_PT)], bidx_v)

    def blk(i, carry):
        pltpu.sync_copy(xm2_hbm.at[pl.ds((w * ROWS_N_PT + i) * 128, 128)],
                        rows_v)
        pltpu.sync_copy(rows_v, acc_sh.at[bidx_v.at[i]], add=True)
        return carry

    lax.fori_loop(0, ROWS_N_PT, blk, 0)
    plsc.subcore_barrier()

    @pl.when(s == 0)
    def _():
        pltpu.sync_copy(acc_sh, out_hbm.at[c])


_pool = pl.kernel(
    _pool_body,
    out_type=jax.ShapeDtypeStruct((NC, GP, H), jnp.float32),
    mesh=_MESH,
    compiler_params=_SC_PARAMS,
    scratch_types=[
        pltpu.VMEM((ROWS_N_PT, 128), jnp.int32),
        pltpu.VMEM((128, H), jnp.float32),
        pltpu.VMEM_SHARED((GP, H), jnp.float32),
    ],
)


R = 2048
GRID = N_PAD // R
_f32 = jnp.float32


def _dense_a_body(x_ref, w_ref, dg0_ref, dg1_ref,
                  h0_ref, h1_ref, v0_ref, v1_ref):
    m = jnp.dot(x_ref[...], w_ref[...], preferred_element_type=_f32)
    v0 = lax.rsqrt(dg0_ref[...] + 1.0)
    v1 = lax.rsqrt(dg1_ref[...] + 1.0)
    h0_ref[...] = m[:, :H] * v0
    h1_ref[...] = m[:, H:] * v1
    v0_ref[...] = v0
    v1_ref[...] = v1


_dense_a = pl.pallas_call(
    _dense_a_body,
    grid=(GRID,),
    in_specs=[
        pl.BlockSpec((R, D_IN), lambda i: (i, 0)),
        pl.BlockSpec((D_IN, 2 * H), lambda i: (0, 0)),
        pl.BlockSpec((R, 1), lambda i: (i, 0)),
        pl.BlockSpec((R, 1), lambda i: (i, 0)),
    ],
    out_specs=[
        pl.BlockSpec((R, H), lambda i: (i, 0)),
        pl.BlockSpec((R, H), lambda i: (i, 0)),
        pl.BlockSpec((R, 1), lambda i: (i, 0)),
        pl.BlockSpec((R, 1), lambda i: (i, 0)),
    ],
    out_shape=[
        jax.ShapeDtypeStruct((N_PAD, H), _f32),
        jax.ShapeDtypeStruct((N_PAD, H), _f32),
        jax.ShapeDtypeStruct((N_PAD, 1), _f32),
        jax.ShapeDtypeStruct((N_PAD, 1), _f32),
    ],
)


def _dense_b_body(s0_ref, s1_ref, h0p_ref, h1p_ref, v0_ref, v1_ref,
                  b10_ref, b11_ref, m1a_ref, m1b_ref, mb1_ref,
                  m1w2_ref, mb2_ref, w20_ref, w21_ref,
                  g0_ref, g1_ref):
    v0 = v0_ref[...]
    v1 = v1_ref[...]
    h0 = jnp.maximum(v0 * (s0_ref[...] + h0p_ref[...]) + b10_ref[...], 0.0)
    h1 = jnp.maximum(v1 * (s1_ref[...] + h1p_ref[...]) + b11_ref[...], 0.0)
    t = jnp.maximum(
        jnp.dot(h0, m1a_ref[...], preferred_element_type=_f32)
        + jnp.dot(h1, m1b_ref[...], preferred_element_type=_f32)
        + mb1_ref[...], 0.0)
    xm = jnp.dot(t, m1w2_ref[...], preferred_element_type=_f32) + mb2_ref[...]
    g0_ref[...] = jnp.dot(xm, w20_ref[...], preferred_element_type=_f32) * v0
    g1_ref[...] = jnp.dot(xm, w21_ref[...], preferred_element_type=_f32) * v1


def _row_spec():
    return pl.BlockSpec((R, H), lambda i: (i, 0))


def _one_spec():
    return pl.BlockSpec((R, 1), lambda i: (i, 0))


def _mat_spec():
    return pl.BlockSpec((H, H), lambda i: (0, 0))


def _bias_spec():
    return pl.BlockSpec((1, H), lambda i: (0, 0))


_dense_b = pl.pallas_call(
    _dense_b_body,
    grid=(GRID,),
    in_specs=[
        _row_spec(), _row_spec(), _row_spec(), _row_spec(),
        _one_spec(), _one_spec(),
        _bias_spec(), _bias_spec(),
        _mat_spec(), _mat_spec(), _bias_spec(),
        _mat_spec(), _bias_spec(),
        _mat_spec(), _mat_spec(),
    ],
    out_specs=[_row_spec(), _row_spec()],
    out_shape=[
        jax.ShapeDtypeStruct((N_PAD, H), _f32),
        jax.ShapeDtypeStruct((N_PAD, H), _f32),
    ],
)


def _dense_c_body(s0_ref, s1_ref, g0p_ref, g1p_ref, v0_ref, v1_ref,
                  b20_ref, b21_ref, m2a_ref, m2b_ref, mb1_ref,
                  m2w2_ref, mb2_ref, xm2_ref):
    g0 = jnp.maximum(v0_ref[...] * (s0_ref[...] + g0p_ref[...]) + b20_ref[...],
                     0.0)
    g1 = jnp.maximum(v1_ref[...] * (s1_ref[...] + g1p_ref[...]) + b21_ref[...],
                     0.0)
    t = jnp.maximum(
        jnp.dot(g0, m2a_ref[...], preferred_element_type=_f32)
        + jnp.dot(g1, m2b_ref[...], preferred_element_type=_f32)
        + mb1_ref[...], 0.0)
    xm2_ref[...] = (jnp.dot(t, m2w2_ref[...], preferred_element_type=_f32)
                    + mb2_ref[...])


_dense_c = pl.pallas_call(
    _dense_c_body,
    grid=(GRID,),
    in_specs=[
        _row_spec(), _row_spec(), _row_spec(), _row_spec(),
        _one_spec(), _one_spec(),
        _bias_spec(), _bias_spec(),
        _mat_spec(), _mat_spec(), _bias_spec(),
        _mat_spec(), _bias_spec(),
    ],
    out_specs=[_row_spec()],
    out_shape=[jax.ShapeDtypeStruct((N_PAD, H), _f32)],
)


def _dense_d_body(p_ref, w_ref, b_ref, o_ref):
    p = p_ref[0, :G, :] + p_ref[1, :G, :]
    o_ref[...] = jnp.dot(p, w_ref[...], preferred_element_type=_f32) + b_ref[...]


_dense_d = pl.pallas_call(
    _dense_d_body,
    grid=(1,),
    in_specs=[
        pl.BlockSpec((NC, GP, H), lambda i: (0, 0, 0)),
        pl.BlockSpec((H, 1), lambda i: (0, 0)),
        pl.BlockSpec((1, 1), lambda i: (0, 0)),
    ],
    out_specs=pl.BlockSpec((G, 1), lambda i: (0, 0)),
    out_shape=jax.ShapeDtypeStruct((G, 1), _f32),
)


def kernel(x, edge_index_0, edge_index_1, index_0, index_1, batch,
           W1_0, b1_0, W1_1, b1_1, W2_0, b2_0, W2_1, b2_1,
           mlp1_W1, mlp1_b1, mlp1_W2, mlp1_b2,
           mlp2_W1, mlp2_b1, mlp2_W2, mlp2_b2,
           lin_W, lin_b):
    i32 = jnp.int32
    x_p = jnp.zeros((N_PAD, D_IN), _f32).at[:N].set(x)
    pad_idx = jnp.full((E_PAD - E,), N_PAD - 1, i32)
    s0 = jnp.concatenate([edge_index_0[0].astype(i32), pad_idx])
    d0 = jnp.concatenate([edge_index_0[1].astype(i32), pad_idx])
    s1 = jnp.concatenate([edge_index_1[0].astype(i32), pad_idx])
    d1 = jnp.concatenate([edge_index_1[1].astype(i32), pad_idx])
    src_all = jnp.stack([s0, s1 + N_PAD])
    dst_all = jnp.stack([d0, d1])
    batch_p = jnp.concatenate(
        [batch.astype(i32), jnp.full((N_PAD - N,), G, i32)]).reshape(ROWS_N, 128)
    zeros_nh = jnp.zeros((N_PAD, H), _f32)
    zeros_n = jnp.zeros((N_PAD,), _f32)
    zeros_g = jnp.zeros((GP, H), _f32)

    deg = _hist(dst_all, zeros_n)
    h0p, h1p, dv0, dv1 = _dense_a(
        x_p, jnp.concatenate([W1_0, W1_1], axis=1),
        deg[0].reshape(N_PAD, 1), deg[1].reshape(N_PAD, 1))

    hs = jnp.concatenate([h0p, h1p], axis=0)
    sc1 = _conv(src_all, dst_all, hs, zeros_nh)

    g0p, g1p = _dense_b(
        sc1[0], sc1[1], h0p, h1p, dv0, dv1,
        b1_0.reshape(1, H), b1_1.reshape(1, H),
        mlp1_W1[:H], mlp1_W1[H:], mlp1_b1.reshape(1, H),
        mlp1_W2, mlp1_b2.reshape(1, H),
        W2_0, W2_1)

    gs = jnp.concatenate([g0p, g1p], axis=0)
    sc2 = _conv(src_all, dst_all, gs, zeros_nh)

    (xm2,) = _dense_c(
        sc2[0], sc2[1], g0p, g1p, dv0, dv1,
        b2_0.reshape(1, H), b2_1.reshape(1, H),
        mlp2_W1[:H], mlp2_W1[H:], mlp2_b1.reshape(1, H),
        mlp2_W2, mlp2_b2.reshape(1, H))

    pooled = _pool(xm2, batch_p, zeros_g)
    out = _dense_d(pooled, lin_W, lin_b.reshape(1, 1))
    return out.reshape(G)


# speedup vs baseline: 70.1715x; 1.6901x over previous
import jax
import jax.numpy as jnp
from jax import lax
from jax.experimental import pallas as pl
from jax.experimental.pallas import tpu as pltpu
from jax.experimental.pallas import tpu_sc as plsc

N = 50000
E = 1600000
D_IN = 128
H = 32
G = 128

NC, NS = 2, 16
N_PAD = 50176
EPT = E // NS
GP = 136
RPT = N_PAD // NS

CB = 256
FB = 390
TB = 160

HB = 2048
FHB = 48
THB = 1696

_MESH = plsc.VectorSubcoreMesh(core_axis_name="c", subcore_axis_name="s")
_SC_PARAMS = pltpu.CompilerParams(use_tc_tiling_on_sc=False)
_f32 = jnp.float32


def _hist_side(ei, ebase, idx_v, idx_t, ones_v, acc_sh):
    def blk(i, carry):
        pltpu.sync_copy(ei.at[1, pl.ds(ebase + i * HB, HB)], idx_v)
        pltpu.sync_copy(ones_v, acc_sh.at[idx_v], add=True)
        return carry

    lax.fori_loop(0, FHB, blk, 0)
    pltpu.sync_copy(ei.at[1, pl.ds(ebase + FHB * HB, THB)], idx_t)
    pltpu.sync_copy(ones_v.at[pl.ds(0, THB)], acc_sh.at[idx_t], add=True)


def _hist_body(ei0, ei1, zeros_hbm, out_hbm, idx_v, idx_t, ones_v, acc_sh):
    c = lax.axis_index("c")
    s = lax.axis_index("s")

    def fill(i, carry):
        ones_v[pl.ds(i * 16, 16)] = jnp.full((16,), 1.0, _f32)
        return carry

    lax.fori_loop(0, HB // 16, fill, 0)
    pltpu.sync_copy(zeros_hbm.at[pl.ds(s * RPT, RPT)],
                    acc_sh.at[pl.ds(s * RPT, RPT)])
    plsc.subcore_barrier()
    ebase = s * EPT

    @pl.when(c == 0)
    def _():
        _hist_side(ei0, ebase, idx_v, idx_t, ones_v, acc_sh)

    @pl.when(c == 1)
    def _():
        _hist_side(ei1, ebase, idx_v, idx_t, ones_v, acc_sh)

    plsc.subcore_barrier()
    pltpu.sync_copy(acc_sh.at[pl.ds(s * RPT, RPT)],
                    out_hbm.at[c, pl.ds(s * RPT, RPT)])


_hist = pl.kernel(
    _hist_body,
    out_type=jax.ShapeDtypeStruct((NC, N_PAD), _f32),
    mesh=_MESH,
    compiler_params=_SC_PARAMS,
    scratch_types=[
        pltpu.VMEM((HB,), jnp.int32),
        pltpu.VMEM((THB,), jnp.int32),
        pltpu.VMEM((HB,), _f32),
        pltpu.VMEM_SHARED((N_PAD,), _f32),
    ],
)


def _conv_side(ei, h_hbm, ebase, idx, idx_t, rows, gsem, ssem, tsem,
               acc_sh):
    def stage(b, buf):
        pltpu.sync_copy(ei.at[:, pl.ds(ebase + b * CB, CB)], idx[buf])

    def gather_start(buf):
        pltpu.make_async_copy(h_hbm.at[idx[buf].at[0]], rows[buf],
                              gsem[buf]).start()

    def gather_wait(buf):
        pltpu.make_async_copy(h_hbm.at[idx[buf].at[0]], rows[buf],
                              gsem[buf]).wait()

    def scatter_start(buf):
        pltpu.async_copy(rows[buf], acc_sh.at[idx[buf].at[1]], ssem[buf],
                         add=True)

    def scatter_wait(buf):
        pltpu.make_async_copy(rows[buf], acc_sh.at[idx[buf].at[1]],
                              ssem[buf]).wait()

    stage(0, 0)
    gather_start(0)
    stage(1, 1)
    gather_start(1)

    def blk(i, carry):
        for half in range(3):
            b = 3 * i + half
            buf = half
            pbuf = (half + 2) % 3

            gather_wait(buf)
            scatter_start(buf)
            p = b + 2

            @pl.when(jnp.logical_and(p < FB, b >= 1))
            def _():
                scatter_wait(pbuf)

            @pl.when(p < FB)
            def _():
                stage(p, pbuf)
                gather_start(pbuf)
        return carry

    lax.fori_loop(0, FB // 3, blk, 0)
    for buf in range(3):
        scatter_wait(buf)
    rt = rows[0].at[pl.ds(0, TB)]
    pltpu.sync_copy(ei.at[:, pl.ds(ebase + FB * CB, TB)], idx_t)
    pltpu.make_async_copy(h_hbm.at[idx_t.at[0]], rt, tsem).start()
    pltpu.make_async_copy(h_hbm.at[idx_t.at[0]], rt, tsem).wait()
    pltpu.sync_copy(rt, acc_sh.at[idx_t.at[1]], add=True)


def _conv_body(ei0, ei1, h0_hbm, h1_hbm, zeros_hbm, out_hbm,
               idx0, idx1, idx2, idx_t, rows0, rows1, rows2,
               gsem0, gsem1, gsem2, ssem0, ssem1, ssem2, tsem, acc_sh):
    c = lax.axis_index("c")
    s = lax.axis_index("s")
    pltpu.sync_copy(zeros_hbm.at[pl.ds(s * RPT, RPT)],
                    acc_sh.at[pl.ds(s * RPT, RPT)])
    plsc.subcore_barrier()
    ebase = s * EPT
    idx = (idx0, idx1, idx2)
    rows = (rows0, rows1, rows2)
    gsem = (gsem0, gsem1, gsem2)
    ssem = (ssem0, ssem1, ssem2)

    @pl.when(c == 0)
    def _():
        _conv_side(ei0, h0_hbm, ebase, idx, idx_t, rows,
                   gsem, ssem, tsem, acc_sh)

    @pl.when(c == 1)
    def _():
        _conv_side(ei1, h1_hbm, ebase, idx, idx_t, rows,
                   gsem, ssem, tsem, acc_sh)

    plsc.subcore_barrier()
    pltpu.sync_copy(acc_sh.at[pl.ds(s * RPT, RPT)],
                    out_hbm.at[c, pl.ds(s * RPT, RPT)])


_conv = pl.kernel(
    _conv_body,
    out_type=jax.ShapeDtypeStruct((NC, N_PAD, H), _f32),
    mesh=_MESH,
    compiler_params=_SC_PARAMS,
    scratch_types=[
        pltpu.VMEM((2, CB), jnp.int32),
        pltpu.VMEM((2, CB), jnp.int32),
        pltpu.VMEM((2, CB), jnp.int32),
        pltpu.VMEM((2, TB), jnp.int32),
        pltpu.VMEM((CB, H), _f32),
        pltpu.VMEM((CB, H), _f32),
        pltpu.VMEM((CB, H), _f32),
        pltpu.SemaphoreType.DMA,
        pltpu.SemaphoreType.DMA,
        pltpu.SemaphoreType.DMA,
        pltpu.SemaphoreType.DMA,
        pltpu.SemaphoreType.DMA,
        pltpu.SemaphoreType.DMA,
        pltpu.SemaphoreType.DMA,
        pltpu.VMEM_SHARED((N_PAD, H), _f32),
    ],
)


ROWS_N = N_PAD // 128


def _pool_body(xm2_hbm, batch_hbm, zeros_hbm, out_hbm, bidx_v, rows_v, acc_sh):
    c = lax.axis_index("c")
    s = lax.axis_index("s")
    w = s * NC + c
    base = 12 * w + jnp.minimum(w, 8)
    nrows = jnp.where(w < 8, 13, 12)

    @pl.when(s == 0)
    def _():
        pltpu.sync_copy(zeros_hbm, acc_sh)

    plsc.subcore_barrier()
    pltpu.sync_copy(batch_hbm.at[pl.ds(base, 13)], bidx_v)

    def blk(i, carry):
        pltpu.sync_copy(xm2_hbm.at[pl.ds((base + i) * 128, 128)], rows_v)
        pltpu.sync_copy(rows_v, acc_sh.at[bidx_v.at[i]], add=True)
        return carry

    lax.fori_loop(0, nrows, blk, 0)
    plsc.subcore_barrier()

    @pl.when(s == 0)
    def _():
        pltpu.sync_copy(acc_sh, out_hbm.at[c])


_pool = pl.kernel(
    _pool_body,
    out_type=jax.ShapeDtypeStruct((NC, GP, H), _f32),
    mesh=_MESH,
    compiler_params=_SC_PARAMS,
    scratch_types=[
        pltpu.VMEM((13, 128), jnp.int32),
        pltpu.VMEM((128, H), _f32),
        pltpu.VMEM_SHARED((GP, H), _f32),
    ],
)


R = 1792
GRID = N_PAD // R


def _dense_a_body(x_ref, w_ref, dg0_ref, dg1_ref,
                  h0_ref, h1_ref, v0_ref, v1_ref):
    m = jnp.dot(x_ref[...], w_ref[...], preferred_element_type=_f32)
    v0 = lax.rsqrt(dg0_ref[...] + 1.0)
    v1 = lax.rsqrt(dg1_ref[...] + 1.0)
    h0_ref[...] = m[:, :H] * v0
    h1_ref[...] = m[:, H:] * v1
    v0_ref[...] = v0
    v1_ref[...] = v1


_dense_a = pl.pallas_call(
    _dense_a_body,
    grid=(GRID,),
    in_specs=[
        pl.BlockSpec((R, D_IN), lambda i: (i, 0)),
        pl.BlockSpec((D_IN, 2 * H), lambda i: (0, 0)),
        pl.BlockSpec((R, 1), lambda i: (i, 0)),
        pl.BlockSpec((R, 1), lambda i: (i, 0)),
    ],
    out_specs=[
        pl.BlockSpec((R, H), lambda i: (i, 0)),
        pl.BlockSpec((R, H), lambda i: (i, 0)),
        pl.BlockSpec((R, 1), lambda i: (i, 0)),
        pl.BlockSpec((R, 1), lambda i: (i, 0)),
    ],
    out_shape=[
        jax.ShapeDtypeStruct((N_PAD, H), _f32),
        jax.ShapeDtypeStruct((N_PAD, H), _f32),
        jax.ShapeDtypeStruct((N_PAD, 1), _f32),
        jax.ShapeDtypeStruct((N_PAD, 1), _f32),
    ],
)


def _dense_b_body(s0_ref, s1_ref, h0p_ref, h1p_ref, v0_ref, v1_ref,
                  b10_ref, b11_ref, m1a_ref, m1b_ref, mb1_ref,
                  m1w2_ref, mb2_ref, w20_ref, w21_ref,
                  g0_ref, g1_ref):
    v0 = v0_ref[...]
    v1 = v1_ref[...]
    h0 = jnp.maximum(v0 * (s0_ref[0] + h0p_ref[...]) + b10_ref[...], 0.0)
    h1 = jnp.maximum(v1 * (s1_ref[0] + h1p_ref[...]) + b11_ref[...], 0.0)
    t = jnp.maximum(
        jnp.dot(h0, m1a_ref[...], preferred_element_type=_f32)
        + jnp.dot(h1, m1b_ref[...], preferred_element_type=_f32)
        + mb1_ref[...], 0.0)
    xm = jnp.dot(t, m1w2_ref[...], preferred_element_type=_f32) + mb2_ref[...]
    g0_ref[...] = jnp.dot(xm, w20_ref[...], preferred_element_type=_f32) * v0
    g1_ref[...] = jnp.dot(xm, w21_ref[...], preferred_element_type=_f32) * v1


def _row_spec():
    return pl.BlockSpec((R, H), lambda i: (i, 0))


def _sc_spec(j):
    return pl.BlockSpec((1, R, H), lambda i, j=j: (j, i, 0))


def _one_spec():
    return pl.BlockSpec((R, 1), lambda i: (i, 0))


def _mat_spec():
    return pl.BlockSpec((H, H), lambda i: (0, 0))


def _bias_spec():
    return pl.BlockSpec((1, H), lambda i: (0, 0))


_dense_b = pl.pallas_call(
    _dense_b_body,
    grid=(GRID,),
    in_specs=[
        _sc_spec(0), _sc_spec(1), _row_spec(), _row_spec(),
        _one_spec(), _one_spec(),
        _bias_spec(), _bias_spec(),
        _mat_spec(), _mat_spec(), _bias_spec(),
        _mat_spec(), _bias_spec(),
        _mat_spec(), _mat_spec(),
    ],
    out_specs=[_row_spec(), _row_spec()],
    out_shape=[
        jax.ShapeDtypeStruct((N_PAD, H), _f32),
        jax.ShapeDtypeStruct((N_PAD, H), _f32),
    ],
)


def _dense_c_body(s0_ref, s1_ref, g0p_ref, g1p_ref, v0_ref, v1_ref,
                  b20_ref, b21_ref, m2a_ref, m2b_ref, mb1_ref,
                  m2w2_ref, mb2_ref, xm2_ref):
    g0 = jnp.maximum(v0_ref[...] * (s0_ref[0] + g0p_ref[...]) + b20_ref[...],
                     0.0)
    g1 = jnp.maximum(v1_ref[...] * (s1_ref[0] + g1p_ref[...]) + b21_ref[...],
                     0.0)
    t = jnp.maximum(
        jnp.dot(g0, m2a_ref[...], preferred_element_type=_f32)
        + jnp.dot(g1, m2b_ref[...], preferred_element_type=_f32)
        + mb1_ref[...], 0.0)
    xm2_ref[...] = (jnp.dot(t, m2w2_ref[...], preferred_element_type=_f32)
                    + mb2_ref[...])


_dense_c = pl.pallas_call(
    _dense_c_body,
    grid=(GRID,),
    in_specs=[
        _sc_spec(0), _sc_spec(1), _row_spec(), _row_spec(),
        _one_spec(), _one_spec(),
        _bias_spec(), _bias_spec(),
        _mat_spec(), _mat_spec(), _bias_spec(),
        _mat_spec(), _bias_spec(),
    ],
    out_specs=[_row_spec()],
    out_shape=[jax.ShapeDtypeStruct((N_PAD, H), _f32)],
)


def _dense_d_body(p_ref, w_ref, b_ref, o_ref):
    p = p_ref[0, :G, :] + p_ref[1, :G, :]
    o_ref[...] = jnp.dot(p, w_ref[...], preferred_element_type=_f32) + b_ref[...]


_dense_d = pl.pallas_call(
    _dense_d_body,
    grid=(1,),
    in_specs=[
        pl.BlockSpec((NC, GP, H), lambda i: (0, 0, 0)),
        pl.BlockSpec((H, 1), lambda i: (0, 0)),
        pl.BlockSpec((1, 1), lambda i: (0, 0)),
    ],
    out_specs=pl.BlockSpec((G, 1), lambda i: (0, 0)),
    out_shape=jax.ShapeDtypeStruct((G, 1), _f32),
)


def kernel(x, edge_index_0, edge_index_1, index_0, index_1, batch,
           W1_0, b1_0, W1_1, b1_1, W2_0, b2_0, W2_1, b2_1,
           mlp1_W1, mlp1_b1, mlp1_W2, mlp1_b2,
           mlp2_W1, mlp2_b1, mlp2_W2, mlp2_b2,
           lin_W, lin_b):
    i32 = jnp.int32
    ei0 = edge_index_0.astype(i32)
    ei1 = edge_index_1.astype(i32)
    x_p = jnp.zeros((N_PAD, D_IN), _f32).at[:N].set(x)
    batch_p = jnp.concatenate(
        [batch.astype(i32), jnp.full((51200 - N,), G, i32)]).reshape(400, 128)
    zeros_nh = jnp.zeros((N_PAD, H), _f32)
    zeros_n = jnp.zeros((N_PAD,), _f32)
    zeros_g = jnp.zeros((GP, H), _f32)

    deg = _hist(ei0, ei1, zeros_n)
    h0p, h1p, dv0, dv1 = _dense_a(
        x_p, jnp.concatenate([W1_0, W1_1], axis=1),
        deg[0].reshape(N_PAD, 1), deg[1].reshape(N_PAD, 1))

    sc1 = _conv(ei0, ei1, h0p, h1p, zeros_nh)

    g0p, g1p = _dense_b(
        sc1, sc1, h0p, h1p, dv0, dv1,
        b1_0.reshape(1, H), b1_1.reshape(1, H),
        mlp1_W1[:H], mlp1_W1[H:], mlp1_b1.reshape(1, H),
        mlp1_W2, mlp1_b2.reshape(1, H),
        W2_0, W2_1)

    sc2 = _conv(ei0, ei1, g0p, g1p, zeros_nh)

    (xm2,) = _dense_c(
        sc2, sc2, g0p, g1p, dv0, dv1,
        b2_0.reshape(1, H), b2_1.reshape(1, H),
        mlp2_W1[:H], mlp2_W1[H:], mlp2_b1.reshape(1, H),
        mlp2_W2, mlp2_b2.reshape(1, H))

    pooled = _pool(xm2, batch_p, zeros_g)
    out = _dense_d(pooled, lin_W, lin_b.reshape(1, 1))
    return out.reshape(G)


# speedup vs baseline: 70.2575x; 1.0012x over previous
import jax
import jax.numpy as jnp
from jax import lax
from jax.experimental import pallas as pl
from jax.experimental.pallas import tpu as pltpu
from jax.experimental.pallas import tpu_sc as plsc

N = 50000
E = 1600000
D_IN = 128
H = 32
G = 128

NC, NS = 2, 16
N_PAD = 50176
EPT = E // NS
GP = 136
RPT = N_PAD // NS

CB = 256
FB = 390
TB = 160

HB = 2048
FHB = 48
THB = 1696

_MESH = plsc.VectorSubcoreMesh(core_axis_name="c", subcore_axis_name="s")
_SC_PARAMS = pltpu.CompilerParams(use_tc_tiling_on_sc=False)
_f32 = jnp.float32


def _hist_side(ei, ebase, idx0, idx1, idx_t, ones_v, hsem0, hsem1, acc_sh):
    idxs = (idx0, idx1)
    hsem = (hsem0, hsem1)

    def stage(b, buf):
        pltpu.sync_copy(ei.at[1, pl.ds(ebase + b * HB, HB)], idxs[buf])

    def sc_start(buf):
        pltpu.async_copy(ones_v, acc_sh.at[idxs[buf]], hsem[buf], add=True)

    def sc_wait(buf):
        pltpu.make_async_copy(ones_v, acc_sh.at[idxs[buf]], hsem[buf]).wait()

    stage(0, 0)

    def blk(i, carry):
        for half in range(2):
            b = 2 * i + half
            buf = half
            nbuf = 1 - half
            sc_start(buf)
            sc_wait(buf)

            @pl.when(b + 1 < FHB)
            def _():
                stage(b + 1, nbuf)
        return carry

    lax.fori_loop(0, FHB // 2, blk, 0)
    pltpu.sync_copy(ei.at[1, pl.ds(ebase + FHB * HB, THB)], idx_t)
    pltpu.sync_copy(ones_v.at[pl.ds(0, THB)], acc_sh.at[idx_t], add=True)


def _hist_body(ei0, ei1, zeros_hbm, out_hbm, idx0, idx1, idx_t, ones_v,
               hsem0, hsem1, acc_sh):
    c = lax.axis_index("c")
    s = lax.axis_index("s")

    def fill(i, carry):
        ones_v[pl.ds(i * 16, 16)] = jnp.full((16,), 1.0, _f32)
        return carry

    lax.fori_loop(0, HB // 16, fill, 0)
    pltpu.sync_copy(zeros_hbm.at[pl.ds(s * RPT, RPT)],
                    acc_sh.at[pl.ds(s * RPT, RPT)])
    plsc.subcore_barrier()
    ebase = s * EPT

    @pl.when(c == 0)
    def _():
        _hist_side(ei0, ebase, idx0, idx1, idx_t, ones_v, hsem0, hsem1, acc_sh)

    @pl.when(c == 1)
    def _():
        _hist_side(ei1, ebase, idx0, idx1, idx_t, ones_v, hsem0, hsem1, acc_sh)

    plsc.subcore_barrier()
    pltpu.sync_copy(acc_sh.at[pl.ds(s * RPT, RPT)],
                    out_hbm.at[c, pl.ds(s * RPT, RPT)])


_hist = pl.kernel(
    _hist_body,
    out_type=jax.ShapeDtypeStruct((NC, N_PAD), _f32),
    mesh=_MESH,
    compiler_params=_SC_PARAMS,
    scratch_types=[
        pltpu.VMEM((HB,), jnp.int32),
        pltpu.VMEM((HB,), jnp.int32),
        pltpu.VMEM((THB,), jnp.int32),
        pltpu.VMEM((HB,), _f32),
        pltpu.SemaphoreType.DMA,
        pltpu.SemaphoreType.DMA,
        pltpu.VMEM_SHARED((N_PAD,), _f32),
    ],
)


def _conv_side(ei, h_hbm, ebase, idx, idx_t, rows, isem, gsem, ssem, tsem,
               acc_sh):
    def stage_start(b, buf):
        pltpu.make_async_copy(ei.at[:, pl.ds(ebase + b * CB, CB)], idx[buf],
                              isem[buf]).start()

    def stage_wait(b, buf):
        pltpu.make_async_copy(ei.at[:, pl.ds(ebase + b * CB, CB)], idx[buf],
                              isem[buf]).wait()

    def gather_start(buf):
        pltpu.make_async_copy(h_hbm.at[idx[buf].at[0]], rows[buf],
                              gsem[buf]).start()

    def gather_wait(buf):
        pltpu.make_async_copy(h_hbm.at[idx[buf].at[0]], rows[buf],
                              gsem[buf]).wait()

    def scatter_start(buf):
        pltpu.async_copy(rows[buf], acc_sh.at[idx[buf].at[1]], ssem[buf],
                         add=True)

    def scatter_wait(buf):
        pltpu.make_async_copy(rows[buf], acc_sh.at[idx[buf].at[1]],
                              ssem[buf]).wait()

    stage_start(0, 0)
    stage_start(1, 1)
    stage_wait(0, 0)
    gather_start(0)

    def blk(i, carry):
        for half in range(3):
            b = 3 * i + half
            buf = half
            nbuf = (half + 1) % 3
            pbuf = (half + 2) % 3

            @pl.when(b + 1 < FB)
            def _():
                stage_wait(b + 1, nbuf)
                gather_start(nbuf)

            gather_wait(buf)

            @pl.when(b >= 1)
            def _():
                scatter_wait(pbuf)

            scatter_start(buf)

            @pl.when(b + 2 < FB)
            def _():
                stage_start(b + 2, pbuf)
        return carry

    lax.fori_loop(0, FB // 3, blk, 0)
    scatter_wait((FB - 1) % 3)
    rt = rows[0].at[pl.ds(0, TB)]
    pltpu.sync_copy(ei.at[:, pl.ds(ebase + FB * CB, TB)], idx_t)
    pltpu.make_async_copy(h_hbm.at[idx_t.at[0]], rt, tsem).start()
    pltpu.make_async_copy(h_hbm.at[idx_t.at[0]], rt, tsem).wait()
    pltpu.sync_copy(rt, acc_sh.at[idx_t.at[1]], add=True)


def _conv_body(ei0, ei1, h0_hbm, h1_hbm, zeros_hbm, out_hbm,
               idx0, idx1, idx2, idx_t, rows0, rows1, rows2,
               isem0, isem1, isem2,
               gsem0, gsem1, gsem2, ssem0, ssem1, ssem2, tsem, acc_sh):
    c = lax.axis_index("c")
    s = lax.axis_index("s")
    pltpu.sync_copy(zeros_hbm.at[pl.ds(s * RPT, RPT)],
                    acc_sh.at[pl.ds(s * RPT, RPT)])
    plsc.subcore_barrier()
    ebase = s * EPT
    idx = (idx0, idx1, idx2)
    rows = (rows0, rows1, rows2)
    isem = (isem0, isem1, isem2)
    gsem = (gsem0, gsem1, gsem2)
    ssem = (ssem0, ssem1, ssem2)

    @pl.when(c == 0)
    def _():
        _conv_side(ei0, h0_hbm, ebase, idx, idx_t, rows,
                   isem, gsem, ssem, tsem, acc_sh)

    @pl.when(c == 1)
    def _():
        _conv_side(ei1, h1_hbm, ebase, idx, idx_t, rows,
                   isem, gsem, ssem, tsem, acc_sh)

    plsc.subcore_barrier()
    pltpu.sync_copy(acc_sh.at[pl.ds(s * RPT, RPT)],
                    out_hbm.at[c, pl.ds(s * RPT, RPT)])


_conv = pl.kernel(
    _conv_body,
    out_type=jax.ShapeDtypeStruct((NC, N_PAD, H), _f32),
    mesh=_MESH,
    compiler_params=_SC_PARAMS,
    scratch_types=[
        pltpu.VMEM((2, CB), jnp.int32),
        pltpu.VMEM((2, CB), jnp.int32),
        pltpu.VMEM((2, CB), jnp.int32),
        pltpu.VMEM((2, TB), jnp.int32),
        pltpu.VMEM((CB, H), _f32),
        pltpu.VMEM((CB, H), _f32),
        pltpu.VMEM((CB, H), _f32),
        pltpu.SemaphoreType.DMA,
        pltpu.SemaphoreType.DMA,
        pltpu.SemaphoreType.DMA,
        pltpu.SemaphoreType.DMA,
        pltpu.SemaphoreType.DMA,
        pltpu.SemaphoreType.DMA,
        pltpu.SemaphoreType.DMA,
        pltpu.SemaphoreType.DMA,
        pltpu.SemaphoreType.DMA,
        pltpu.SemaphoreType.DMA,
        pltpu.VMEM_SHARED((N_PAD, H), _f32),
    ],
)


ROWS_N = N_PAD // 128


def _pool_body(xm2_hbm, batch_hbm, zeros_hbm, out_hbm, bidx_v, rows_v, acc_sh):
    c = lax.axis_index("c")
    s = lax.axis_index("s")
    w = s * NC + c
    base = 12 * w + jnp.minimum(w, 8)
    nrows = jnp.where(w < 8, 13, 12)

    @pl.when(s == 0)
    def _():
        pltpu.sync_copy(zeros_hbm, acc_sh)

    plsc.subcore_barrier()
    pltpu.sync_copy(batch_hbm.at[pl.ds(base, 13)], bidx_v)

    def blk(i, carry):
        pltpu.sync_copy(xm2_hbm.at[pl.ds((base + i) * 128, 128)], rows_v)
        pltpu.sync_copy(rows_v, acc_sh.at[bidx_v.at[i]], add=True)
        return carry

    lax.fori_loop(0, nrows, blk, 0)
    plsc.subcore_barrier()

    @pl.when(s == 0)
    def _():
        pltpu.sync_copy(acc_sh, out_hbm.at[c])


_pool = pl.kernel(
    _pool_body,
    out_type=jax.ShapeDtypeStruct((NC, GP, H), _f32),
    mesh=_MESH,
    compiler_params=_SC_PARAMS,
    scratch_types=[
        pltpu.VMEM((13, 128), jnp.int32),
        pltpu.VMEM((128, H), _f32),
        pltpu.VMEM_SHARED((GP, H), _f32),
    ],
)


R = 3584
GRID = N_PAD // R


def _dense_a_body(x_ref, w_ref, dg0_ref, dg1_ref,
                  h0_ref, h1_ref, v0_ref, v1_ref):
    m = jnp.dot(x_ref[...], w_ref[...], preferred_element_type=_f32)
    v0 = lax.rsqrt(dg0_ref[...] + 1.0)
    v1 = lax.rsqrt(dg1_ref[...] + 1.0)
    h0_ref[...] = m[:, :H] * v0
    h1_ref[...] = m[:, H:] * v1
    v0_ref[...] = v0
    v1_ref[...] = v1


_dense_a = pl.pallas_call(
    _dense_a_body,
    grid=(GRID,),
    in_specs=[
        pl.BlockSpec((R, D_IN), lambda i: (i, 0)),
        pl.BlockSpec((D_IN, 2 * H), lambda i: (0, 0)),
        pl.BlockSpec((R, 1), lambda i: (i, 0)),
        pl.BlockSpec((R, 1), lambda i: (i, 0)),
    ],
    out_specs=[
        pl.BlockSpec((R, H), lambda i: (i, 0)),
        pl.BlockSpec((R, H), lambda i: (i, 0)),
        pl.BlockSpec((R, 1), lambda i: (i, 0)),
        pl.BlockSpec((R, 1), lambda i: (i, 0)),
    ],
    out_shape=[
        jax.ShapeDtypeStruct((N_PAD, H), _f32),
        jax.ShapeDtypeStruct((N_PAD, H), _f32),
        jax.ShapeDtypeStruct((N_PAD, 1), _f32),
        jax.ShapeDtypeStruct((N_PAD, 1), _f32),
    ],
)


def _dense_b_body(s0_ref, s1_ref, h0p_ref, h1p_ref, v0_ref, v1_ref,
                  b10_ref, b11_ref, m1a_ref, m1b_ref, mb1_ref,
                  m1w2_ref, mb2_ref, w20_ref, w21_ref,
                  g0_ref, g1_ref):
    v0 = v0_ref[...]
    v1 = v1_ref[...]
    h0 = jnp.maximum(v0 * (s0_ref[0] + h0p_ref[...]) + b10_ref[...], 0.0)
    h1 = jnp.maximum(v1 * (s1_ref[0] + h1p_ref[...]) + b11_ref[...], 0.0)
    t = jnp.maximum(
        jnp.dot(h0, m1a_ref[...], preferred_element_type=_f32)
        + jnp.dot(h1, m1b_ref[...], preferred_element_type=_f32)
        + mb1_ref[...], 0.0)
    xm = jnp.dot(t, m1w2_ref[...], preferred_element_type=_f32) + mb2_ref[...]
    g0_ref[...] = jnp.dot(xm, w20_ref[...], preferred_element_type=_f32) * v0
    g1_ref[...] = jnp.dot(xm, w21_ref[...], preferred_element_type=_f32) * v1


def _row_spec():
    return pl.BlockSpec((R, H), lambda i: (i, 0))


def _sc_spec(j):
    return pl.BlockSpec((1, R, H), lambda i, j=j: (j, i, 0))


def _one_spec():
    return pl.BlockSpec((R, 1), lambda i: (i, 0))


def _mat_spec():
    return pl.BlockSpec((H, H), lambda i: (0, 0))


def _bias_spec():
    return pl.BlockSpec((1, H), lambda i: (0, 0))


_dense_b = pl.pallas_call(
    _dense_b_body,
    grid=(GRID,),
    in_specs=[
        _sc_spec(0), _sc_spec(1), _row_spec(), _row_spec(),
        _one_spec(), _one_spec(),
        _bias_spec(), _bias_spec(),
        _mat_spec(), _mat_spec(), _bias_spec(),
        _mat_spec(), _bias_spec(),
        _mat_spec(), _mat_spec(),
    ],
    out_specs=[_row_spec(), _row_spec()],
    out_shape=[
        jax.ShapeDtypeStruct((N_PAD, H), _f32),
        jax.ShapeDtypeStruct((N_PAD, H), _f32),
    ],
)


def _dense_c_body(s0_ref, s1_ref, g0p_ref, g1p_ref, v0_ref, v1_ref,
                  b20_ref, b21_ref, m2a_ref, m2b_ref, mb1_ref,
                  m2w2_ref, mb2_ref, xm2_ref):
    g0 = jnp.maximum(v0_ref[...] * (s0_ref[0] + g0p_ref[...]) + b20_ref[...],
                     0.0)
    g1 = jnp.maximum(v1_ref[...] * (s1_ref[0] + g1p_ref[...]) + b21_ref[...],
                     0.0)
    t = jnp.maximum(
        jnp.dot(g0, m2a_ref[...], preferred_element_type=_f32)
        + jnp.dot(g1, m2b_ref[...], preferred_element_type=_f32)
        + mb1_ref[...], 0.0)
    xm2_ref[...] = (jnp.dot(t, m2w2_ref[...], preferred_element_type=_f32)
                    + mb2_ref[...])


_dense_c = pl.pallas_call(
    _dense_c_body,
    grid=(GRID,),
    in_specs=[
        _sc_spec(0), _sc_spec(1), _row_spec(), _row_spec(),
        _one_spec(), _one_spec(),
        _bias_spec(), _bias_spec(),
        _mat_spec(), _mat_spec(), _bias_spec(),
        _mat_spec(), _bias_spec(),
    ],
    out_specs=[_row_spec()],
    out_shape=[jax.ShapeDtypeStruct((N_PAD, H), _f32)],
)


def _dense_d_body(p_ref, w_ref, b_ref, o_ref):
    p = p_ref[0, :G, :] + p_ref[1, :G, :]
    o_ref[...] = jnp.dot(p, w_ref[...], preferred_element_type=_f32) + b_ref[...]


_dense_d = pl.pallas_call(
    _dense_d_body,
    grid=(1,),
    in_specs=[
        pl.BlockSpec((NC, GP, H), lambda i: (0, 0, 0)),
        pl.BlockSpec((H, 1), lambda i: (0, 0)),
        pl.BlockSpec((1, 1), lambda i: (0, 0)),
    ],
    out_specs=pl.BlockSpec((G, 1), lambda i: (0, 0)),
    out_shape=jax.ShapeDtypeStruct((G, 1), _f32),
)


def kernel(x, edge_index_0, edge_index_1, index_0, index_1, batch,
           W1_0, b1_0, W1_1, b1_1, W2_0, b2_0, W2_1, b2_1,
           mlp1_W1, mlp1_b1, mlp1_W2, mlp1_b2,
           mlp2_W1, mlp2_b1, mlp2_W2, mlp2_b2,
           lin_W, lin_b):
    i32 = jnp.int32
    ei0 = edge_index_0.astype(i32)
    ei1 = edge_index_1.astype(i32)
    x_p = jnp.zeros((N_PAD, D_IN), _f32).at[:N].set(x)
    batch_p = jnp.concatenate(
        [batch.astype(i32), jnp.full((51200 - N,), G, i32)]).reshape(400, 128)
    zeros_nh = jnp.zeros((N_PAD, H), _f32)
    zeros_n = jnp.zeros((N_PAD,), _f32)
    zeros_g = jnp.zeros((GP, H), _f32)

    deg = _hist(ei0, ei1, zeros_n)
    h0p, h1p, dv0, dv1 = _dense_a(
        x_p, jnp.concatenate([W1_0, W1_1], axis=1),
        deg[0].reshape(N_PAD, 1), deg[1].reshape(N_PAD, 1))

    sc1 = _conv(ei0, ei1, h0p, h1p, zeros_nh)

    g0p, g1p = _dense_b(
        sc1, sc1, h0p, h1p, dv0, dv1,
        b1_0.reshape(1, H), b1_1.reshape(1, H),
        mlp1_W1[:H], mlp1_W1[H:], mlp1_b1.reshape(1, H),
        mlp1_W2, mlp1_b2.reshape(1, H),
        W2_0, W2_1)

    sc2 = _conv(ei0, ei1, g0p, g1p, zeros_nh)

    (xm2,) = _dense_c(
        sc2, sc2, g0p, g1p, dv0, dv1,
        b2_0.reshape(1, H), b2_1.reshape(1, H),
        mlp2_W1[:H], mlp2_W1[H:], mlp2_b1.reshape(1, H),
        mlp2_W2, mlp2_b2.reshape(1, H))

    pooled = _pool(xm2, batch_p, zeros_g)
    out = _dense_d(pooled, lin_W, lin_b.reshape(1, 1))
    return out.reshape(G)


# speedup vs baseline: 71.2079x; 1.0135x over previous
import jax
import jax.numpy as jnp
from jax import lax
from jax.experimental import pallas as pl
from jax.experimental.pallas import tpu as pltpu
from jax.experimental.pallas import tpu_sc as plsc

N = 50000
E = 1600000
D_IN = 128
H = 32
G = 128

NC, NS = 2, 16
N_PAD = 50176
EPT = E // NS
GP = 136
RPT = N_PAD // NS

HB = 2048
FHB = 48
THB = 1696

_MESH = plsc.VectorSubcoreMesh(core_axis_name="c", subcore_axis_name="s")
_SC_PARAMS = pltpu.CompilerParams(use_tc_tiling_on_sc=False)
_f32 = jnp.float32


def _hist_side(ei, ebase, idx0, idx1, idx_t, ones_v, hsem0, hsem1, acc_sh):
    idxs = (idx0, idx1)
    hsem = (hsem0, hsem1)

    def stage(b, buf):
        pltpu.sync_copy(ei.at[1, pl.ds(ebase + b * HB, HB)], idxs[buf])

    def sc_start(buf):
        pltpu.async_copy(ones_v, acc_sh.at[idxs[buf]], hsem[buf], add=True)

    def sc_wait(buf):
        pltpu.make_async_copy(ones_v, acc_sh.at[idxs[buf]], hsem[buf]).wait()

    stage(0, 0)

    def blk(i, carry):
        for half in range(2):
            b = 2 * i + half
            buf = half
            nbuf = 1 - half
            sc_start(buf)
            sc_wait(buf)

            @pl.when(b + 1 < FHB)
            def _():
                stage(b + 1, nbuf)
        return carry

    lax.fori_loop(0, FHB // 2, blk, 0)
    pltpu.sync_copy(ei.at[1, pl.ds(ebase + FHB * HB, THB)], idx_t)
    pltpu.sync_copy(ones_v.at[pl.ds(0, THB)], acc_sh.at[idx_t], add=True)


def _hist_body(ei0, ei1, zeros_hbm, out_hbm, idx0, idx1, idx_t, ones_v,
               hsem0, hsem1, acc_sh):
    c = lax.axis_index("c")
    s = lax.axis_index("s")

    def fill(i, carry):
        ones_v[pl.ds(i * 16, 16)] = jnp.full((16,), 1.0, _f32)
        return carry

    lax.fori_loop(0, HB // 16, fill, 0)
    pltpu.sync_copy(zeros_hbm.at[pl.ds(s * RPT, RPT)],
                    acc_sh.at[pl.ds(s * RPT, RPT)])
    plsc.subcore_barrier()
    ebase = s * EPT

    @pl.when(c == 0)
    def _():
        _hist_side(ei0, ebase, idx0, idx1, idx_t, ones_v, hsem0, hsem1, acc_sh)

    @pl.when(c == 1)
    def _():
        _hist_side(ei1, ebase, idx0, idx1, idx_t, ones_v, hsem0, hsem1, acc_sh)

    plsc.subcore_barrier()
    pltpu.sync_copy(acc_sh.at[pl.ds(s * RPT, RPT)],
                    out_hbm.at[c, pl.ds(s * RPT, RPT)])


_hist = pl.kernel(
    _hist_body,
    out_type=jax.ShapeDtypeStruct((NC, N_PAD), _f32),
    mesh=_MESH,
    compiler_params=_SC_PARAMS,
    scratch_types=[
        pltpu.VMEM((HB,), jnp.int32),
        pltpu.VMEM((HB,), jnp.int32),
        pltpu.VMEM((THB,), jnp.int32),
        pltpu.VMEM((HB,), _f32),
        pltpu.SemaphoreType.DMA,
        pltpu.SemaphoreType.DMA,
        pltpu.VMEM_SHARED((N_PAD,), _f32),
    ],
)


CB = 192
FB = 520
TB = 160

NB = 4


def _conv_side(ei, h_hbm, ebase, idx, idx_t, rows, isem, gsem, ssem, tsem,
               acc_sh):
    def stage_start(b, buf):
        pltpu.make_async_copy(ei.at[:, pl.ds(ebase + b * CB, CB)], idx[buf],
                              isem[buf]).start()

    def stage_wait(b, buf):
        pltpu.make_async_copy(ei.at[:, pl.ds(ebase + b * CB, CB)], idx[buf],
                              isem[buf]).wait()

    def gather_start(buf):
        pltpu.make_async_copy(h_hbm.at[idx[buf].at[0]], rows[buf],
                              gsem[buf]).start()

    def gather_wait(buf):
        pltpu.make_async_copy(h_hbm.at[idx[buf].at[0]], rows[buf],
                              gsem[buf]).wait()

    def scatter_start(buf):
        pltpu.async_copy(rows[buf], acc_sh.at[idx[buf].at[1]], ssem[buf],
                         add=True)

    def scatter_wait(buf):
        pltpu.make_async_copy(rows[buf], acc_sh.at[idx[buf].at[1]],
                              ssem[buf]).wait()

    stage_start(0, 0)
    stage_start(1, 1)
    stage_start(2, 2)
    stage_wait(0, 0)
    gather_start(0)
    stage_wait(1, 1)
    gather_start(1)

    def blk(i, carry):
        for half in range(NB):
            b = NB * i + half
            buf = half

            @pl.when(b + 2 < FB)
            def _():
                stage_wait(b + 2, (half + 2) % NB)
                gather_start((half + 2) % NB)

            gather_wait(buf)

            @pl.when(b >= 1)
            def _():
                scatter_wait((half + NB - 1) % NB)

            scatter_start(buf)

            @pl.when(b + 3 < FB)
            def _():
                stage_start(b + 3, (half + 3) % NB)
        return carry

    lax.fori_loop(0, FB // NB, blk, 0)
    scatter_wait((FB - 1) % NB)
    rt = rows[0].at[pl.ds(0, TB)]
    pltpu.sync_copy(ei.at[:, pl.ds(ebase + FB * CB, TB)], idx_t)
    pltpu.make_async_copy(h_hbm.at[idx_t.at[0]], rt, tsem).start()
    pltpu.make_async_copy(h_hbm.at[idx_t.at[0]], rt, tsem).wait()
    pltpu.sync_copy(rt, acc_sh.at[idx_t.at[1]], add=True)


def _conv_body(ei0, ei1, h0_hbm, h1_hbm, zeros_hbm, out_hbm,
               idx0, idx1, idx2, idx3, idx_t, rows0, rows1, rows2, rows3,
               isem0, isem1, isem2, isem3,
               gsem0, gsem1, gsem2, gsem3,
               ssem0, ssem1, ssem2, ssem3, tsem, acc_sh):
    c = lax.axis_index("c")
    s = lax.axis_index("s")
    pltpu.sync_copy(zeros_hbm.at[pl.ds(s * RPT, RPT)],
                    acc_sh.at[pl.ds(s * RPT, RPT)])
    plsc.subcore_barrier()
    ebase = s * EPT
    idx = (idx0, idx1, idx2, idx3)
    rows = (rows0, rows1, rows2, rows3)
    isem = (isem0, isem1, isem2, isem3)
    gsem = (gsem0, gsem1, gsem2, gsem3)
    ssem = (ssem0, ssem1, ssem2, ssem3)

    @pl.when(c == 0)
    def _():
        _conv_side(ei0, h0_hbm, ebase, idx, idx_t, rows,
                   isem, gsem, ssem, tsem, acc_sh)

    @pl.when(c == 1)
    def _():
        _conv_side(ei1, h1_hbm, ebase, idx, idx_t, rows,
                   isem, gsem, ssem, tsem, acc_sh)

    plsc.subcore_barrier()
    pltpu.sync_copy(acc_sh.at[pl.ds(s * RPT, RPT)],
                    out_hbm.at[c, pl.ds(s * RPT, RPT)])


_conv = pl.kernel(
    _conv_body,
    out_type=jax.ShapeDtypeStruct((NC, N_PAD, H), _f32),
    mesh=_MESH,
    compiler_params=_SC_PARAMS,
    scratch_types=(
        [pltpu.VMEM((2, CB), jnp.int32)] * NB
        + [pltpu.VMEM((2, TB), jnp.int32)]
        + [pltpu.VMEM((CB, H), _f32)] * NB
        + [pltpu.SemaphoreType.DMA] * (3 * NB + 1)
        + [pltpu.VMEM_SHARED((N_PAD, H), _f32)]
    ),
)


ROWS_N = N_PAD // 128


def _pool_body(xm2_hbm, batch_hbm, zeros_hbm, out_hbm, bidx_v, rows_v, acc_sh):
    c = lax.axis_index("c")
    s = lax.axis_index("s")
    w = s * NC + c
    base = 12 * w + jnp.minimum(w, 8)
    nrows = jnp.where(w < 8, 13, 12)

    @pl.when(s == 0)
    def _():
        pltpu.sync_copy(zeros_hbm, acc_sh)

    plsc.subcore_barrier()
    pltpu.sync_copy(batch_hbm.at[pl.ds(base, 13)], bidx_v)

    def blk(i, carry):
        pltpu.sync_copy(xm2_hbm.at[pl.ds((base + i) * 128, 128)], rows_v)
        pltpu.sync_copy(rows_v, acc_sh.at[bidx_v.at[i]], add=True)
        return carry

    lax.fori_loop(0, nrows, blk, 0)
    plsc.subcore_barrier()

    @pl.when(s == 0)
    def _():
        pltpu.sync_copy(acc_sh, out_hbm.at[c])


_pool = pl.kernel(
    _pool_body,
    out_type=jax.ShapeDtypeStruct((NC, GP, H), _f32),
    mesh=_MESH,
    compiler_params=_SC_PARAMS,
    scratch_types=[
        pltpu.VMEM((13, 128), jnp.int32),
        pltpu.VMEM((128, H), _f32),
        pltpu.VMEM_SHARED((GP, H), _f32),
    ],
)


R = 3584
GRID = N_PAD // R


def _dense_a_body(x_ref, w_ref, dg0_ref, dg1_ref,
                  h0_ref, h1_ref, v0_ref, v1_ref):
    m = jnp.dot(x_ref[...], w_ref[...], preferred_element_type=_f32)
    v0 = lax.rsqrt(dg0_ref[...] + 1.0)
    v1 = lax.rsqrt(dg1_ref[...] + 1.0)
    h0_ref[...] = m[:, :H] * v0
    h1_ref[...] = m[:, H:] * v1
    v0_ref[...] = v0
    v1_ref[...] = v1


_dense_a = pl.pallas_call(
    _dense_a_body,
    grid=(GRID,),
    in_specs=[
        pl.BlockSpec((R, D_IN), lambda i: (i, 0)),
        pl.BlockSpec((D_IN, 2 * H), lambda i: (0, 0)),
        pl.BlockSpec((R, 1), lambda i: (i, 0)),
        pl.BlockSpec((R, 1), lambda i: (i, 0)),
    ],
    out_specs=[
        pl.BlockSpec((R, H), lambda i: (i, 0)),
        pl.BlockSpec((R, H), lambda i: (i, 0)),
        pl.BlockSpec((R, 1), lambda i: (i, 0)),
        pl.BlockSpec((R, 1), lambda i: (i, 0)),
    ],
    out_shape=[
        jax.ShapeDtypeStruct((N_PAD, H), _f32),
        jax.ShapeDtypeStruct((N_PAD, H), _f32),
        jax.ShapeDtypeStruct((N_PAD, 1), _f32),
        jax.ShapeDtypeStruct((N_PAD, 1), _f32),
    ],
)


def _dense_b_body(s0_ref, s1_ref, h0p_ref, h1p_ref, v0_ref, v1_ref,
                  b10_ref, b11_ref, m1a_ref, m1b_ref, mb1_ref,
                  m1w2_ref, mb2_ref, w20_ref, w21_ref,
                  g0_ref, g1_ref):
    v0 = v0_ref[...]
    v1 = v1_ref[...]
    h0 = jnp.maximum(v0 * (s0_ref[0] + h0p_ref[...]) + b10_ref[...], 0.0)
    h1 = jnp.maximum(v1 * (s1_ref[0] + h1p_ref[...]) + b11_ref[...], 0.0)
    t = jnp.maximum(
        jnp.dot(h0, m1a_ref[...], preferred_element_type=_f32)
        + jnp.dot(h1, m1b_ref[...], preferred_element_type=_f32)
        + mb1_ref[...], 0.0)
    xm = jnp.dot(t, m1w2_ref[...], preferred_element_type=_f32) + mb2_ref[...]
    g0_ref[...] = jnp.dot(xm, w20_ref[...], preferred_element_type=_f32) * v0
    g1_ref[...] = jnp.dot(xm, w21_ref[...], preferred_element_type=_f32) * v1


def _row_spec():
    return pl.BlockSpec((R, H), lambda i: (i, 0))


def _sc_spec(j):
    return pl.BlockSpec((1, R, H), lambda i, j=j: (j, i, 0))


def _one_spec():
    return pl.BlockSpec((R, 1), lambda i: (i, 0))


def _mat_spec():
    return pl.BlockSpec((H, H), lambda i: (0, 0))


def _bias_spec():
    return pl.BlockSpec((1, H), lambda i: (0, 0))


_dense_b = pl.pallas_call(
    _dense_b_body,
    grid=(GRID,),
    in_specs=[
        _sc_spec(0), _sc_spec(1), _row_spec(), _row_spec(),
        _one_spec(), _one_spec(),
        _bias_spec(), _bias_spec(),
        _mat_spec(), _mat_spec(), _bias_spec(),
        _mat_spec(), _bias_spec(),
        _mat_spec(), _mat_spec(),
    ],
    out_specs=[_row_spec(), _row_spec()],
    out_shape=[
        jax.ShapeDtypeStruct((N_PAD, H), _f32),
        jax.ShapeDtypeStruct((N_PAD, H), _f32),
    ],
)


def _dense_c_body(s0_ref, s1_ref, g0p_ref, g1p_ref, v0_ref, v1_ref,
                  b20_ref, b21_ref, m2a_ref, m2b_ref, mb1_ref,
                  m2w2_ref, mb2_ref, xm2_ref):
    g0 = jnp.maximum(v0_ref[...] * (s0_ref[0] + g0p_ref[...]) + b20_ref[...],
                     0.0)
    g1 = jnp.maximum(v1_ref[...] * (s1_ref[0] + g1p_ref[...]) + b21_ref[...],
                     0.0)
    t = jnp.maximum(
        jnp.dot(g0, m2a_ref[...], preferred_element_type=_f32)
        + jnp.dot(g1, m2b_ref[...], preferred_element_type=_f32)
        + mb1_ref[...], 0.0)
    xm2_ref[...] = (jnp.dot(t, m2w2_ref[...], preferred_element_type=_f32)
                    + mb2_ref[...])


_dense_c = pl.pallas_call(
    _dense_c_body,
    grid=(GRID,),
    in_specs=[
        _sc_spec(0), _sc_spec(1), _row_spec(), _row_spec(),
        _one_spec(), _one_spec(),
        _bias_spec(), _bias_spec(),
        _mat_spec(), _mat_spec(), _bias_spec(),
        _mat_spec(), _bias_spec(),
    ],
    out_specs=[_row_spec()],
    out_shape=[jax.ShapeDtypeStruct((N_PAD, H), _f32)],
)


def _dense_d_body(p_ref, w_ref, b_ref, o_ref):
    p = p_ref[0, :G, :] + p_ref[1, :G, :]
    o_ref[...] = jnp.dot(p, w_ref[...], preferred_element_type=_f32) + b_ref[...]


_dense_d = pl.pallas_call(
    _dense_d_body,
    grid=(1,),
    in_specs=[
        pl.BlockSpec((NC, GP, H), lambda i: (0, 0, 0)),
        pl.BlockSpec((H, 1), lambda i: (0, 0)),
        pl.BlockSpec((1, 1), lambda i: (0, 0)),
    ],
    out_specs=pl.BlockSpec((G, 1), lambda i: (0, 0)),
    out_shape=jax.ShapeDtypeStruct((G, 1), _f32),
)


def kernel(x, edge_index_0, edge_index_1, index_0, index_1, batch,
           W1_0, b1_0, W1_1, b1_1, W2_0, b2_0, W2_1, b2_1,
           mlp1_W1, mlp1_b1, mlp1_W2, mlp1_b2,
           mlp2_W1, mlp2_b1, mlp2_W2, mlp2_b2,
           lin_W, lin_b):
    i32 = jnp.int32
    ei0 = edge_index_0.astype(i32)
    ei1 = edge_index_1.astype(i32)
    x_p = jnp.zeros((N_PAD, D_IN), _f32).at[:N].set(x)
    batch_p = jnp.concatenate(
        [batch.astype(i32), jnp.full((51200 - N,), G, i32)]).reshape(400, 128)
    zeros_nh = jnp.zeros((N_PAD, H), _f32)
    zeros_n = jnp.zeros((N_PAD,), _f32)
    zeros_g = jnp.zeros((GP, H), _f32)

    deg = _hist(ei0, ei1, zeros_n)
    h0p, h1p, dv0, dv1 = _dense_a(
        x_p, jnp.concatenate([W1_0, W1_1], axis=1),
        deg[0].reshape(N_PAD, 1), deg[1].reshape(N_PAD, 1))

    sc1 = _conv(ei0, ei1, h0p, h1p, zeros_nh)

    g0p, g1p = _dense_b(
        sc1, sc1, h0p, h1p, dv0, dv1,
        b1_0.reshape(1, H), b1_1.reshape(1, H),
        mlp1_W1[:H], mlp1_W1[H:], mlp1_b1.reshape(1, H),
        mlp1_W2, mlp1_b2.reshape(1, H),
        W2_0, W2_1)

    sc2 = _conv(ei0, ei1, g0p, g1p, zeros_nh)

    (xm2,) = _dense_c(
        sc2, sc2, g0p, g1p, dv0, dv1,
        b2_0.reshape(1, H), b2_1.reshape(1, H),
        mlp2_W1[:H], mlp2_W1[H:], mlp2_b1.reshape(1, H),
        mlp2_W2, mlp2_b2.reshape(1, H))

    pooled = _pool(xm2, batch_p, zeros_g)
    out = _dense_d(pooled, lin_W, lin_b.reshape(1, 1))
    return out.reshape(G)


# speedup vs baseline: 71.4561x; 1.0035x over previous
import jax
import jax.numpy as jnp
from jax import lax
from jax.experimental import pallas as pl
from jax.experimental.pallas import tpu as pltpu
from jax.experimental.pallas import tpu_sc as plsc

N = 50000
E = 1600000
D_IN = 128
H = 32
G = 128

NC, NS = 2, 16
N_PAD = 50176
EPT = E // NS
GP = 136
RPT = N_PAD // NS

HB = 2048
FHB = 48
THB = 1696

_MESH = plsc.VectorSubcoreMesh(core_axis_name="c", subcore_axis_name="s")
_SC_PARAMS = pltpu.CompilerParams(use_tc_tiling_on_sc=False)
_f32 = jnp.float32


def _hist_side(ei, ebase, idx0, idx1, idx_t, ones_v, hsem0, hsem1, acc_sh):
    idxs = (idx0, idx1)
    hsem = (hsem0, hsem1)

    def stage(b, buf):
        pltpu.sync_copy(ei.at[1, pl.ds(ebase + b * HB, HB)], idxs[buf])

    def sc_start(buf):
        pltpu.async_copy(ones_v, acc_sh.at[idxs[buf]], hsem[buf], add=True)

    def sc_wait(buf):
        pltpu.make_async_copy(ones_v, acc_sh.at[idxs[buf]], hsem[buf]).wait()

    stage(0, 0)

    def blk(i, carry):
        for half in range(2):
            b = 2 * i + half
            buf = half
            nbuf = 1 - half
            sc_start(buf)
            sc_wait(buf)

            @pl.when(b + 1 < FHB)
            def _():
                stage(b + 1, nbuf)
        return carry

    lax.fori_loop(0, FHB // 2, blk, 0)
    pltpu.sync_copy(ei.at[1, pl.ds(ebase + FHB * HB, THB)], idx_t)
    pltpu.sync_copy(ones_v.at[pl.ds(0, THB)], acc_sh.at[idx_t], add=True)


def _hist_body(ei0, ei1, zeros_hbm, out_hbm, idx0, idx1, idx_t, ones_v,
               hsem0, hsem1, acc_sh):
    c = lax.axis_index("c")
    s = lax.axis_index("s")

    def fill(i, carry):
        ones_v[pl.ds(i * 16, 16)] = jnp.full((16,), 1.0, _f32)
        return carry

    lax.fori_loop(0, HB // 16, fill, 0)
    pltpu.sync_copy(zeros_hbm.at[pl.ds(s * RPT, RPT)],
                    acc_sh.at[pl.ds(s * RPT, RPT)])
    plsc.subcore_barrier()
    ebase = s * EPT

    @pl.when(c == 0)
    def _():
        _hist_side(ei0, ebase, idx0, idx1, idx_t, ones_v, hsem0, hsem1, acc_sh)

    @pl.when(c == 1)
    def _():
        _hist_side(ei1, ebase, idx0, idx1, idx_t, ones_v, hsem0, hsem1, acc_sh)

    plsc.subcore_barrier()
    pltpu.sync_copy(acc_sh.at[pl.ds(s * RPT, RPT)],
                    out_hbm.at[c, pl.ds(s * RPT, RPT)])


_hist = pl.kernel(
    _hist_body,
    out_type=jax.ShapeDtypeStruct((NC, N_PAD), _f32),
    mesh=_MESH,
    compiler_params=_SC_PARAMS,
    scratch_types=[
        pltpu.VMEM((HB,), jnp.int32),
        pltpu.VMEM((HB,), jnp.int32),
        pltpu.VMEM((THB,), jnp.int32),
        pltpu.VMEM((HB,), _f32),
        pltpu.SemaphoreType.DMA,
        pltpu.SemaphoreType.DMA,
        pltpu.VMEM_SHARED((N_PAD,), _f32),
    ],
)


CB = 192
FB = 520
TB = 160

NB = 4


def _conv_side(ei, h_hbm, ebase, idx, idx_t, rows, isem, gsem, ssem, tsem,
               acc_sh):
    def stage_start(b, buf):
        pltpu.make_async_copy(ei.at[:, pl.ds(ebase + b * CB, CB)], idx[buf],
                              isem[buf]).start()

    def stage_wait(b, buf):
        pltpu.make_async_copy(ei.at[:, pl.ds(ebase + b * CB, CB)], idx[buf],
                              isem[buf]).wait()

    def gather_start(buf):
        pltpu.make_async_copy(h_hbm.at[idx[buf].at[0]], rows[buf],
                              gsem[buf]).start()

    def gather_wait(buf):
        pltpu.make_async_copy(h_hbm.at[idx[buf].at[0]], rows[buf],
                              gsem[buf]).wait()

    def scatter_start(buf):
        pltpu.async_copy(rows[buf], acc_sh.at[idx[buf].at[1]], ssem[buf],
                         add=True)

    def scatter_wait(buf):
        pltpu.make_async_copy(rows[buf], acc_sh.at[idx[buf].at[1]],
                              ssem[buf]).wait()

    stage_start(0, 0)
    stage_start(1, 1)
    stage_start(2, 2)
    stage_wait(0, 0)
    gather_start(0)
    stage_wait(1, 1)
    gather_start(1)

    def blk(i, carry):
        for half in range(NB):
            b = NB * i + half
            buf = half

            @pl.when(b + 2 < FB)
            def _():
                stage_wait(b + 2, (half + 2) % NB)
                gather_start((half + 2) % NB)

            gather_wait(buf)

            @pl.when(b >= 1)
            def _():
                scatter_wait((half + NB - 1) % NB)

            scatter_start(buf)

            @pl.when(b + 3 < FB)
            def _():
                stage_start(b + 3, (half + 3) % NB)
        return carry

    lax.fori_loop(0, FB // NB, blk, 0)
    scatter_wait((FB - 1) % NB)
    rt = rows[0].at[pl.ds(0, TB)]
    pltpu.sync_copy(ei.at[:, pl.ds(ebase + FB * CB, TB)], idx_t)
    pltpu.make_async_copy(h_hbm.at[idx_t.at[0]], rt, tsem).start()
    pltpu.make_async_copy(h_hbm.at[idx_t.at[0]], rt, tsem).wait()
    pltpu.sync_copy(rt, acc_sh.at[idx_t.at[1]], add=True)


def _conv_body(ei0, ei1, h0_hbm, h1_hbm, zeros_hbm, out_hbm,
               idx0, idx1, idx2, idx3, idx_t, rows0, rows1, rows2, rows3,
               isem0, isem1, isem2, isem3,
               gsem0, gsem1, gsem2, gsem3,
               ssem0, ssem1, ssem2, ssem3, tsem, acc_sh):
    c = lax.axis_index("c")
    s = lax.axis_index("s")
    pltpu.sync_copy(zeros_hbm.at[pl.ds(s * RPT, RPT)],
                    acc_sh.at[pl.ds(s * RPT, RPT)])
    plsc.subcore_barrier()
    ebase = s * EPT
    idx = (idx0, idx1, idx2, idx3)
    rows = (rows0, rows1, rows2, rows3)
    isem = (isem0, isem1, isem2, isem3)
    gsem = (gsem0, gsem1, gsem2, gsem3)
    ssem = (ssem0, ssem1, ssem2, ssem3)

    @pl.when(c == 0)
    def _():
        _conv_side(ei0, h0_hbm, ebase, idx, idx_t, rows,
                   isem, gsem, ssem, tsem, acc_sh)

    @pl.when(c == 1)
    def _():
        _conv_side(ei1, h1_hbm, ebase, idx, idx_t, rows,
                   isem, gsem, ssem, tsem, acc_sh)

    plsc.subcore_barrier()
    pltpu.sync_copy(acc_sh.at[pl.ds(s * RPT, RPT)],
                    out_hbm.at[c, pl.ds(s * RPT, RPT)])


_conv = pl.kernel(
    _conv_body,
    out_type=jax.ShapeDtypeStruct((NC, N_PAD, H), _f32),
    mesh=_MESH,
    compiler_params=_SC_PARAMS,
    scratch_types=(
        [pltpu.VMEM((2, CB), jnp.int32)] * NB
        + [pltpu.VMEM((2, TB), jnp.int32)]
        + [pltpu.VMEM((CB, H), _f32)] * NB
        + [pltpu.SemaphoreType.DMA] * (3 * NB + 1)
        + [pltpu.VMEM_SHARED((N_PAD, H), _f32)]
    ),
)


ROWS_N = N_PAD // 128


def _pool_body(xm2_hbm, batch_hbm, zeros_hbm, out_hbm, bidx_v, rows_v, acc_sh):
    c = lax.axis_index("c")
    s = lax.axis_index("s")
    w = s * NC + c
    base = 12 * w + jnp.minimum(w, 8)
    nrows = jnp.where(w < 8, 13, 12)

    @pl.when(s == 0)
    def _():
        pltpu.sync_copy(zeros_hbm, acc_sh)

    plsc.subcore_barrier()
    pltpu.sync_copy(batch_hbm.at[pl.ds(base, 13)], bidx_v)

    def blk(i, carry):
        pltpu.sync_copy(xm2_hbm.at[pl.ds((base + i) * 128, 128)], rows_v)
        pltpu.sync_copy(rows_v, acc_sh.at[bidx_v.at[i]], add=True)
        return carry

    lax.fori_loop(0, nrows, blk, 0)
    plsc.subcore_barrier()

    @pl.when(s == 0)
    def _():
        pltpu.sync_copy(acc_sh, out_hbm.at[c])


_pool = pl.kernel(
    _pool_body,
    out_type=jax.ShapeDtypeStruct((NC, GP, H), _f32),
    mesh=_MESH,
    compiler_params=_SC_PARAMS,
    scratch_types=[
        pltpu.VMEM((13, 128), jnp.int32),
        pltpu.VMEM((128, H), _f32),
        pltpu.VMEM_SHARED((GP, H), _f32),
    ],
)


R = 3584
GRID = N_PAD // R


def _dense_a1_body(x_ref, w_ref, m_ref):
    m_ref[...] = jnp.dot(x_ref[...], w_ref[...], preferred_element_type=_f32)


_dense_a1 = pl.pallas_call(
    _dense_a1_body,
    grid=(GRID,),
    in_specs=[
        pl.BlockSpec((R, D_IN), lambda i: (i, 0)),
        pl.BlockSpec((D_IN, 2 * H), lambda i: (0, 0)),
    ],
    out_specs=pl.BlockSpec((R, 2 * H), lambda i: (i, 0)),
    out_shape=jax.ShapeDtypeStruct((N_PAD, 2 * H), _f32),
)


def _dense_a2_body(m_ref, dg0_ref, dg1_ref, h0_ref, h1_ref, v0_ref, v1_ref):
    m = m_ref[...]
    v0 = lax.rsqrt(dg0_ref[...] + 1.0)
    v1 = lax.rsqrt(dg1_ref[...] + 1.0)
    h0_ref[...] = m[:, :H] * v0
    h1_ref[...] = m[:, H:] * v1
    v0_ref[...] = v0
    v1_ref[...] = v1


_dense_a2 = pl.pallas_call(
    _dense_a2_body,
    grid=(GRID,),
    in_specs=[
        pl.BlockSpec((R, 2 * H), lambda i: (i, 0)),
        pl.BlockSpec((R, 1), lambda i: (i, 0)),
        pl.BlockSpec((R, 1), lambda i: (i, 0)),
    ],
    out_specs=[
        pl.BlockSpec((R, H), lambda i: (i, 0)),
        pl.BlockSpec((R, H), lambda i: (i, 0)),
        pl.BlockSpec((R, 1), lambda i: (i, 0)),
        pl.BlockSpec((R, 1), lambda i: (i, 0)),
    ],
    out_shape=[
        jax.ShapeDtypeStruct((N_PAD, H), _f32),
        jax.ShapeDtypeStruct((N_PAD, H), _f32),
        jax.ShapeDtypeStruct((N_PAD, 1), _f32),
        jax.ShapeDtypeStruct((N_PAD, 1), _f32),
    ],
)


def _dense_b_body(s0_ref, s1_ref, h0p_ref, h1p_ref, v0_ref, v1_ref,
                  b10_ref, b11_ref, m1a_ref, m1b_ref, mb1_ref,
                  m1w2_ref, mb2_ref, w20_ref, w21_ref,
                  g0_ref, g1_ref):
    v0 = v0_ref[...]
    v1 = v1_ref[...]
    h0 = jnp.maximum(v0 * (s0_ref[0] + h0p_ref[...]) + b10_ref[...], 0.0)
    h1 = jnp.maximum(v1 * (s1_ref[0] + h1p_ref[...]) + b11_ref[...], 0.0)
    t = jnp.maximum(
        jnp.dot(h0, m1a_ref[...], preferred_element_type=_f32)
        + jnp.dot(h1, m1b_ref[...], preferred_element_type=_f32)
        + mb1_ref[...], 0.0)
    xm = jnp.dot(t, m1w2_ref[...], preferred_element_type=_f32) + mb2_ref[...]
    g0_ref[...] = jnp.dot(xm, w20_ref[...], preferred_element_type=_f32) * v0
    g1_ref[...] = jnp.dot(xm, w21_ref[...], preferred_element_type=_f32) * v1


def _row_spec():
    return pl.BlockSpec((R, H), lambda i: (i, 0))


def _sc_spec(j):
    return pl.BlockSpec((1, R, H), lambda i, j=j: (j, i, 0))


def _one_spec():
    return pl.BlockSpec((R, 1), lambda i: (i, 0))


def _mat_spec():
    return pl.BlockSpec((H, H), lambda i: (0, 0))


def _bias_spec():
    return pl.BlockSpec((1, H), lambda i: (0, 0))


_dense_b = pl.pallas_call(
    _dense_b_body,
    grid=(GRID,),
    in_specs=[
        _sc_spec(0), _sc_spec(1), _row_spec(), _row_spec(),
        _one_spec(), _one_spec(),
        _bias_spec(), _bias_spec(),
        _mat_spec(), _mat_spec(), _bias_spec(),
        _mat_spec(), _bias_spec(),
        _mat_spec(), _mat_spec(),
    ],
    out_specs=[_row_spec(), _row_spec()],
    out_shape=[
        jax.ShapeDtypeStruct((N_PAD, H), _f32),
        jax.ShapeDtypeStruct((N_PAD, H), _f32),
    ],
)


def _dense_c_body(s0_ref, s1_ref, g0p_ref, g1p_ref, v0_ref, v1_ref,
                  b20_ref, b21_ref, m2a_ref, m2b_ref, mb1_ref,
                  m2w2_ref, mb2_ref, xm2_ref):
    g0 = jnp.maximum(v0_ref[...] * (s0_ref[0] + g0p_ref[...]) + b20_ref[...],
                     0.0)
    g1 = jnp.maximum(v1_ref[...] * (s1_ref[0] + g1p_ref[...]) + b21_ref[...],
                     0.0)
    t = jnp.maximum(
        jnp.dot(g0, m2a_ref[...], preferred_element_type=_f32)
        + jnp.dot(g1, m2b_ref[...], preferred_element_type=_f32)
        + mb1_ref[...], 0.0)
    xm2_ref[...] = (jnp.dot(t, m2w2_ref[...], preferred_element_type=_f32)
                    + mb2_ref[...])


_dense_c = pl.pallas_call(
    _dense_c_body,
    grid=(GRID,),
    in_specs=[
        _sc_spec(0), _sc_spec(1), _row_spec(), _row_spec(),
        _one_spec(), _one_spec(),
        _bias_spec(), _bias_spec(),
        _mat_spec(), _mat_spec(), _bias_spec(),
        _mat_spec(), _bias_spec(),
    ],
    out_specs=[_row_spec()],
    out_shape=[jax.ShapeDtypeStruct((N_PAD, H), _f32)],
)


def _dense_d_body(p_ref, w_ref, b_ref, o_ref):
    p = p_ref[0, :G, :] + p_ref[1, :G, :]
    o_ref[...] = jnp.dot(p, w_ref[...], preferred_element_type=_f32) + b_ref[...]


_dense_d = pl.pallas_call(
    _dense_d_body,
    grid=(1,),
    in_specs=[
        pl.BlockSpec((NC, GP, H), lambda i: (0, 0, 0)),
        pl.BlockSpec((H, 1), lambda i: (0, 0)),
        pl.BlockSpec((1, 1), lambda i: (0, 0)),
    ],
    out_specs=pl.BlockSpec((G, 1), lambda i: (0, 0)),
    out_shape=jax.ShapeDtypeStruct((G, 1), _f32),
)


def kernel(x, edge_index_0, edge_index_1, index_0, index_1, batch,
           W1_0, b1_0, W1_1, b1_1, W2_0, b2_0, W2_1, b2_1,
           mlp1_W1, mlp1_b1, mlp1_W2, mlp1_b2,
           mlp2_W1, mlp2_b1, mlp2_W2, mlp2_b2,
           lin_W, lin_b):
    i32 = jnp.int32
    ei0 = edge_index_0.astype(i32)
    ei1 = edge_index_1.astype(i32)
    batch_p = jnp.concatenate(
        [batch.astype(i32), jnp.full((51200 - N,), G, i32)]).reshape(400, 128)
    zeros_nh = jnp.zeros((N_PAD, H), _f32)
    zeros_n = jnp.zeros((N_PAD,), _f32)
    zeros_g = jnp.zeros((GP, H), _f32)

    deg = _hist(ei0, ei1, zeros_n)
    m = _dense_a1(x, jnp.concatenate([W1_0, W1_1], axis=1))
    h0p, h1p, dv0, dv1 = _dense_a2(
        m, deg[0].reshape(N_PAD, 1), deg[1].reshape(N_PAD, 1))

    sc1 = _conv(ei0, ei1, h0p, h1p, zeros_nh)

    g0p, g1p = _dense_b(
        sc1, sc1, h0p, h1p, dv0, dv1,
        b1_0.reshape(1, H), b1_1.reshape(1, H),
        mlp1_W1[:H], mlp1_W1[H:], mlp1_b1.reshape(1, H),
        mlp1_W2, mlp1_b2.reshape(1, H),
        W2_0, W2_1)

    sc2 = _conv(ei0, ei1, g0p, g1p, zeros_nh)

    (xm2,) = _dense_c(
        sc2, sc2, g0p, g1p, dv0, dv1,
        b2_0.reshape(1, H), b2_1.reshape(1, H),
        mlp2_W1[:H], mlp2_W1[H:], mlp2_b1.reshape(1, H),
        mlp2_W2, mlp2_b2.reshape(1, H))

    pooled = _pool(xm2, batch_p, zeros_g)
    out = _dense_d(pooled, lin_W, lin_b.reshape(1, 1))
    return out.reshape(G)
